# Initial kernel scaffold; baseline (speedup 1.0000x reference)
import jax
import jax.numpy as jnp
from jax.experimental import pallas as pl


def kernel(input, hg_idx, related_items, label, uid, params):
    raise NotImplementedError("write your pallas kernel here")


# speedup vs baseline: 2.9380x; 2.9380x over previous
import functools

import jax
import jax.numpy as jnp
import numpy as np
from jax import lax
from jax.experimental import pallas as pl
from jax.experimental.pallas import tpu as pltpu
from jax.experimental.pallas import tpu_sc as plsc

BSZ = 64
LENS = 300
EMB = 128
HALF = EMB // 2
D = EMB + HALF
N_NODES = BSZ * LENS
E_INC = 307200
N_ITEMS = 20000
HEADS = 4
DH = D // HEADS

NC, NS = 2, 16
EPT = E_INC // NS
ICH = 128
NCH = EPT // ICH
RNG = N_NODES // NC
NDUM = 512
RWT = RNG // NS
XCH = 120
BPT = RWT // LENS


def _make_pos():
    position = np.arange(LENS)[:, None].astype(np.float64)
    div_term = np.exp(np.arange(0, EMB, 2) * (-np.log(10000.0) / EMB))
    pe = np.zeros((LENS + 1, EMB))
    pe[1:, 0::2] = np.sin(position * div_term)
    pe[1:, 1::2] = np.cos(position * div_term)
    return np.ascontiguousarray((pe[:LENS] * 0.002).astype(np.float32))


_POS = _make_pos()


def _proj_body(tab_ref, w_ref, b_ref, o_ref):
    o_ref[...] = jnp.dot(tab_ref[...], w_ref[...],
                         preferred_element_type=jnp.float32) + b_ref[...]


def _project_table(table, w, b, rows_blk):
    n, k = table.shape
    return pl.pallas_call(
        _proj_body,
        grid=(n // rows_blk,),
        in_specs=[
            pl.BlockSpec((rows_blk, k), lambda i: (i, 0)),
            pl.BlockSpec((k, EMB), lambda i: (0, 0)),
            pl.BlockSpec((1, EMB), lambda i: (0, 0)),
        ],
        out_specs=pl.BlockSpec((rows_blk, EMB), lambda i: (i, 0)),
        out_shape=jax.ShapeDtypeStruct((n, EMB), jnp.float32),
    )(table, w, b.reshape(1, EMB))


_MMBLK = 1200


def _mm2_body(xt_ref, tht_ref, xi_ref, thi_ref, yt_ref, yi_ref):
    yt_ref[...] = jnp.dot(xt_ref[...], tht_ref[...],
                          preferred_element_type=jnp.float32)
    yi_ref[...] = jnp.dot(xi_ref[...], thi_ref[...],
                          preferred_element_type=jnp.float32)


def _mm2_scaled_body(xt_ref, tht_ref, xi_ref, thi_ref, d_ref,
                     bt_ref, bi_ref, yt_ref, yi_ref):
    d = d_ref[...]
    xt = xt_ref[...] * d + bt_ref[...]
    xi = xi_ref[...] * d + bi_ref[...]
    yt_ref[...] = jnp.dot(xt, tht_ref[...], preferred_element_type=jnp.float32)
    yi_ref[...] = jnp.dot(xi, thi_ref[...], preferred_element_type=jnp.float32)


def _layer_matmul(xt, tht, xi, thi, dinv=None, bt=None, bi=None):
    xspec = pl.BlockSpec((_MMBLK, EMB), lambda i: (i, 0))
    tspec = pl.BlockSpec((EMB, EMB), lambda i: (0, 0))
    bspec = pl.BlockSpec((1, EMB), lambda i: (0, 0))
    dspec = pl.BlockSpec((_MMBLK, 1), lambda i: (i, 0))
    out_shape = [jax.ShapeDtypeStruct((N_NODES, EMB), jnp.float32)] * 2
    if dinv is None:
        return pl.pallas_call(
            _mm2_body,
            grid=(N_NODES // _MMBLK,),
            in_specs=[xspec, tspec, xspec, tspec],
            out_specs=[xspec, xspec],
            out_shape=out_shape,
        )(xt, tht, xi, thi)
    return pl.pallas_call(
        _mm2_scaled_body,
        grid=(N_NODES // _MMBLK,),
        in_specs=[xspec, tspec, xspec, tspec, dspec, bspec, bspec],
        out_specs=[xspec, xspec],
        out_shape=out_shape,
    )(xt, tht, xi, thi, dinv.reshape(N_NODES, 1),
      bt.reshape(1, EMB), bi.reshape(1, EMB))


def _escale_body(et_ref, ei_ref, b_ref, ot_ref, oi_ref):
    b = b_ref[...]
    ot_ref[...] = et_ref[...] * b
    oi_ref[...] = ei_ref[...] * b


def _escale(et, ei, binv):
    xspec = pl.BlockSpec((_MMBLK, EMB), lambda i: (i, 0))
    dspec = pl.BlockSpec((_MMBLK, 1), lambda i: (i, 0))
    return pl.pallas_call(
        _escale_body,
        grid=(N_NODES // _MMBLK,),
        in_specs=[xspec, xspec, dspec],
        out_specs=[xspec, xspec],
        out_shape=[jax.ShapeDtypeStruct((N_NODES, EMB), jnp.float32)] * 2,
    )(et, ei, binv.reshape(N_NODES, 1))


def _head_body(tg0_ref, ig0_ref, d0_ref, bt1_ref, bi1_ref,
               user_ref, w3_ref, b3_ref,
               wq_ref, bq_ref, wk_ref, bk_ref, wv_ref, bv_ref,
               wo_ref, bo_ref, wf1_ref, bf1_ref, wf2_ref, bf2_ref,
               ln1g_ref, ln1b_ref, ln2g_ref, ln2b_ref,
               dw_ref, db_ref, w4_ref, b4_ref, out_ref):
    f32 = jnp.float32

    def mm(a, b):
        return jnp.dot(a, b, preferred_element_type=f32)

    def layer_norm(x, g, b):
        m = jnp.mean(x, axis=-1, keepdims=True)
        v = jnp.mean((x - m) * (x - m), axis=-1, keepdims=True)
        return (x - m) / jnp.sqrt(v + 1e-5) * g + b

    d0 = d0_ref[...]
    tg0 = tg0_ref[...] * d0 + bt1_ref[...]
    ig0 = ig0_ref[...] * d0 + bi1_ref[...]
    user = user_ref[...][:, :HALF]
    text_user = jnp.concatenate([tg0, user], axis=1)
    img_user = jnp.concatenate([ig0, user], axis=1)
    tiu = mm(jnp.concatenate([text_user, img_user], axis=1),
             w3_ref[...]) + b3_ref[...]
    q = mm(tiu, wq_ref[...]) + bq_ref[...]
    k1 = mm(text_user, wk_ref[...]) + bk_ref[...]
    k2 = mm(img_user, wk_ref[...]) + bk_ref[...]
    v1 = mm(text_user, wv_ref[...]) + bv_ref[...]
    v2 = mm(img_user, wv_ref[...]) + bv_ref[...]
    scale = np.float32(1.0 / np.sqrt(DH))
    ao_parts = []
    for h in range(HEADS):
        sl = slice(h * DH, (h + 1) * DH)
        qh, k1h, k2h = q[:, sl], k1[:, sl], k2[:, sl]
        s1 = jnp.sum(qh * k1h, axis=1, keepdims=True) * scale
        s2 = jnp.sum(qh * k2h, axis=1, keepdims=True) * scale
        m = jnp.maximum(s1, s2)
        e1 = jnp.exp(s1 - m)
        e2 = jnp.exp(s2 - m)
        tot = e1 + e2
        ao_parts.append((e1 / tot) * v1[:, sl] + (e2 / tot) * v2[:, sl])
    ao = mm(jnp.concatenate(ao_parts, axis=1), wo_ref[...]) + bo_ref[...]
    x = layer_norm(tiu + ao, ln1g_ref[...], ln1b_ref[...])
    ff = mm(jnp.maximum(mm(x, wf1_ref[...]) + bf1_ref[...], 0.0),
            wf2_ref[...]) + bf2_ref[...]
    x = layer_norm(x + ff, ln2g_ref[...], ln2b_ref[...])
    x = jnp.maximum(mm(x, dw_ref[...]) + db_ref[...], 0.0)
    out_ref[...] = mm(x, w4_ref[...]) + b4_ref[...]


def _head(tg0, ig0, d0, bt1, bi1, user, p):
    w4p = jnp.pad(p['W4'], ((0, 0), (0, 127)))
    b4p = jnp.pad(p['b4'], (0, 127)).reshape(1, 128)
    args = [tg0, ig0, d0, bt1.reshape(1, EMB), bi1.reshape(1, EMB), user,
            p['W3'], p['b3'].reshape(1, D),
            p['Wq'], p['bq'].reshape(1, D), p['Wk'], p['bk'].reshape(1, D),
            p['Wv'], p['bv'].reshape(1, D), p['Wo'], p['bo'].reshape(1, D),
            p['Wf1'], p['bf1'].reshape(1, D), p['Wf2'], p['bf2'].reshape(1, D),
            p['ln1_g'].reshape(1, D), p['ln1_b'].reshape(1, D),
            p['ln2_g'].reshape(1, D), p['ln2_b'].reshape(1, D),
            p['dW'], p['db'].reshape(1, D), w4p, b4p]
    out = pl.pallas_call(
        _head_body,
        out_shape=jax.ShapeDtypeStruct((BSZ, 128), jnp.float32),
    )(*args)
    return out[:, :1]


_MESH = plsc.VectorSubcoreMesh(
    core_axis_name="c", subcore_axis_name="s", num_cores=NC, num_subcores=NS)


@functools.partial(
    pl.kernel,
    out_type=[
        jax.ShapeDtypeStruct((N_NODES, EMB), jnp.float32),
        jax.ShapeDtypeStruct((N_NODES, EMB), jnp.float32),
        jax.ShapeDtypeStruct((BSZ, EMB), jnp.float32),
        jax.ShapeDtypeStruct((N_NODES,), jnp.float32),
        jax.ShapeDtypeStruct((N_NODES,), jnp.float32),
    ],
    mesh=_MESH,
    scratch_types=[
        pltpu.VMEM((XCH,), jnp.int32),
        pltpu.VMEM((XCH, EMB), jnp.float32),
        pltpu.VMEM((LENS, EMB), jnp.float32),
        pltpu.VMEM_SHARED((N_NODES,), jnp.float32),
        pltpu.VMEM((RWT * 2,), jnp.float32),
        pltpu.VMEM((ICH,), jnp.float32),
        pltpu.VMEM((ICH,), jnp.int32),
        pltpu.VMEM((8,), jnp.int32),
        pltpu.VMEM((8, EMB), jnp.float32),
        pltpu.SemaphoreType.DMA,
    ],
)
def _sc_prep(pt_hbm, pi_hbm, ri_hbm, node_hbm, edge_hbm, uid_hbm, ut_hbm,
             pos_hbm, xt_hbm, xi_hbm, user_hbm, dinv_hbm, binv_hbm,
             idxv, rows, posv, deg, degv, onesv, div, uidv, urows, sem):
    c = lax.axis_index("c")
    s = lax.axis_index("s")
    w = c * NS + s
    nb = w * (N_NODES // (NC * NS))
    dpt = RWT * 2

    def zdeg(j, _):
        degv[pl.ds(j * 16, 16)] = jnp.zeros((16,), jnp.float32)
        return 0
    lax.fori_loop(0, dpt // 16, zdeg, 0)
    pltpu.sync_copy(degv, deg.at[pl.ds(s * dpt, dpt)])
    for j in range(ICH // 16):
        onesv[pl.ds(j * 16, 16)] = jnp.ones((16,), jnp.float32)
    plsc.subcore_barrier()

    def deg_pass(src_hbm):
        def body(ch, _):
            base = s * EPT + ch * ICH
            pltpu.sync_copy(src_hbm.at[pl.ds(base, ICH)], div)
            pltpu.sync_copy(onesv, deg.at[div], add=True)
            return 0
        lax.fori_loop(0, NCH, body, 0)

    @pl.when(c == 0)
    def _():
        deg_pass(node_hbm)

    @pl.when(c == 1)
    def _():
        deg_pass(edge_hbm)

    pltpu.sync_copy(pos_hbm, posv)
    for p_hbm, x_hbm in ((pt_hbm, xt_hbm), (pi_hbm, xi_hbm)):
        def xbody(k, _):
            pltpu.sync_copy(ri_hbm.at[pl.ds(nb + k * XCH, XCH)], idxv)
            pltpu.async_copy(p_hbm.at[idxv], rows, sem).wait()

            def posadd(r, _):
                pr = lax.rem(k * XCH + r, LENS)
                for j in range(EMB // 16):
                    sl = pl.ds(j * 16, 16)
                    rows[r, sl] = rows[r, sl] + posv[pr, sl]
                return 0
            lax.fori_loop(0, XCH, posadd, 0)
            pltpu.sync_copy(rows, x_hbm.at[pl.ds(nb + k * XCH, XCH)])
            return 0
        lax.fori_loop(0, (N_NODES // (NC * NS)) // XCH, xbody, 0)

    @pl.when((c == 0) & (s < 8))
    def _():
        pltpu.sync_copy(uid_hbm.at[pl.ds(s * 8, 8)], uidv)
        pltpu.async_copy(ut_hbm.at[uidv], urows, sem).wait()
        pltpu.sync_copy(urows, user_hbm.at[pl.ds(s * 8, 8)])

    plsc.subcore_barrier()

    pltpu.sync_copy(deg.at[pl.ds(s * dpt, dpt)], degv)

    def inv(j, _):
        sl = pl.ds(j * 16, 16)
        v = degv[sl]
        degv[sl] = jnp.where(v > 0.0, 1.0 / v, 0.0)
        return 0
    lax.fori_loop(0, dpt // 16, inv, 0)

    @pl.when(c == 0)
    def _():
        pltpu.sync_copy(degv, dinv_hbm.at[pl.ds(s * dpt, dpt)])

    @pl.when(c == 1)
    def _():
        pltpu.sync_copy(degv, binv_hbm.at[pl.ds(s * dpt, dpt)])


@functools.partial(
    pl.kernel,
    out_type=[
        jax.ShapeDtypeStruct((N_NODES, EMB), jnp.float32),
        jax.ShapeDtypeStruct((N_NODES, EMB), jnp.float32),
    ],
    mesh=_MESH,
    scratch_types=[
        pltpu.VMEM_SHARED((RNG + NDUM, EMB), jnp.float32),
        pltpu.VMEM((ICH,), jnp.int32),
        pltpu.VMEM((ICH,), jnp.int32),
        pltpu.VMEM((ICH,), jnp.int32),
        pltpu.VMEM((ICH, EMB), jnp.float32),
        pltpu.VMEM((120, EMB), jnp.float32),
        pltpu.SemaphoreType.DMA,
    ],
)
def _seg_full(gt_hbm, gi_hbm, src_hbm, dst_hbm, ot_hbm, oi_hbm,
              acc, nv, ev, evc, rows, zbuf, sem):
    c = lax.axis_index("c")
    s = lax.axis_index("s")
    lo = c * RNG
    r0 = s * RWT

    def zb(r, _):
        for j in range(EMB // 16):
            zbuf[r, pl.ds(j * 16, 16)] = jnp.zeros((16,), jnp.float32)
        return 0
    lax.fori_loop(0, 120, zb, 0)

    for m, (g_hbm, o_hbm) in enumerate(((gt_hbm, ot_hbm),
                                        (gi_hbm, oi_hbm))):
        for k in range(RWT // 120):
            pltpu.sync_copy(zbuf, acc.at[pl.ds(r0 + k * 120, 120)])
        plsc.subcore_barrier()

        def body(ch, _):
            base = s * EPT + ch * ICH
            pltpu.sync_copy(src_hbm.at[pl.ds(base, ICH)], nv)
            pltpu.async_copy(g_hbm.at[nv], rows, sem).wait()
            pltpu.sync_copy(dst_hbm.at[pl.ds(base, ICH)], ev)
            for j in range(ICH // 16):
                sl = pl.ds(j * 16, 16)
                e = ev[sl]
                loc = e - lo
                dummy = RNG + (e & (NDUM - 1))
                ok = (loc >= 0) & (loc < RNG)
                evc[sl] = jnp.where(ok, loc, dummy)
            pltpu.sync_copy(rows, acc.at[evc], add=True)
            return 0
        lax.fori_loop(0, NCH, body, 0)
        plsc.subcore_barrier()

        pltpu.sync_copy(acc.at[pl.ds(r0, RWT)],
                        o_hbm.at[pl.ds(lo + r0, RWT)])
        if m == 0:
            plsc.subcore_barrier()


def kernel(input, hg_idx, related_items, label, uid, params):
    p = params
    node = hg_idx[0]
    edge = hg_idx[1]

    pt = _project_table(p['text_table'], p['W1'], p['b1'], 400)
    pi = _project_table(p['img_table'], p['W2'], p['b2'], 400)

    pos = jnp.asarray(_POS)
    ut_p = jnp.pad(p['user_table'], ((0, 0), (0, EMB - HALF)))
    xt, xi, user, dinv, binv = _sc_prep(
        pt, pi, related_items, node, edge, uid, ut_p, pos)

    yt, yi = _layer_matmul(xt, p['theta_t0'], xi, p['theta_i0'])
    et_raw, ei_raw = _seg_full(yt, yi, node, edge)
    et, ei = _escale(et_raw, ei_raw, binv)
    xt2, xi2 = _seg_full(et, ei, edge, node)

    yt2, yi2 = _layer_matmul(xt2, p['theta_t1'], xi2, p['theta_i1'],
                             dinv, p['bias_t0'], p['bias_i0'])
    et2_raw, ei2_raw = _seg_full(yt2, yi2, node, edge)
    et2, ei2 = _escale(et2_raw, ei2_raw, binv)
    xt3, xi3 = _seg_full(et2, ei2, edge, node)
    tg0_raw = xt3[::LENS]
    ig0_raw = xi3[::LENS]

    d0 = dinv[::LENS].reshape(BSZ, 1)
    return _head(tg0_raw, ig0_raw, d0, p['bias_t1'], p['bias_i1'], user, p)


# speedup vs baseline: 5.5453x; 1.8875x over previous
import functools

import jax
import jax.numpy as jnp
import numpy as np
from jax import lax
from jax.experimental import pallas as pl
from jax.experimental.pallas import tpu as pltpu
from jax.experimental.pallas import tpu_sc as plsc

BSZ = 64
LENS = 300
EMB = 128
HALF = EMB // 2
D = EMB + HALF
N_NODES = BSZ * LENS
E_INC = 307200
N_ITEMS = 20000
HEADS = 4
DH = D // HEADS

NC, NS = 2, 16
EPT = E_INC // NS
ICH = 128
NCH = EPT // ICH
RNG = N_NODES // NC
NDUM = 512
RWT = RNG // NS
XCH = 120
BPT = RWT // LENS


def _make_pos():
    position = np.arange(LENS)[:, None].astype(np.float64)
    div_term = np.exp(np.arange(0, EMB, 2) * (-np.log(10000.0) / EMB))
    pe = np.zeros((LENS + 1, EMB))
    pe[1:, 0::2] = np.sin(position * div_term)
    pe[1:, 1::2] = np.cos(position * div_term)
    return np.ascontiguousarray((pe[:LENS] * 0.002).astype(np.float32))


_POS = _make_pos()


def _proj_body(tab_ref, w_ref, b_ref, o_ref):
    o_ref[...] = jnp.dot(tab_ref[...], w_ref[...],
                         preferred_element_type=jnp.float32) + b_ref[...]


def _project_table(table, w, b, rows_blk):
    n, k = table.shape
    return pl.pallas_call(
        _proj_body,
        grid=(n // rows_blk,),
        in_specs=[
            pl.BlockSpec((rows_blk, k), lambda i: (i, 0)),
            pl.BlockSpec((k, EMB), lambda i: (0, 0)),
            pl.BlockSpec((1, EMB), lambda i: (0, 0)),
        ],
        out_specs=pl.BlockSpec((rows_blk, EMB), lambda i: (i, 0)),
        out_shape=jax.ShapeDtypeStruct((n, EMB), jnp.float32),
    )(table, w, b.reshape(1, EMB))


_MMBLK = 1200


def _mm2_body(xt_ref, tht_ref, xi_ref, thi_ref, yt_ref, yi_ref):
    yt_ref[...] = jnp.dot(xt_ref[...], tht_ref[...],
                          preferred_element_type=jnp.float32)
    yi_ref[...] = jnp.dot(xi_ref[...], thi_ref[...],
                          preferred_element_type=jnp.float32)


def _mm2_scaled_body(xt_ref, tht_ref, xi_ref, thi_ref, d_ref,
                     bt_ref, bi_ref, yt_ref, yi_ref):
    d = d_ref[...]
    xt = xt_ref[...] * d + bt_ref[...]
    xi = xi_ref[...] * d + bi_ref[...]
    yt_ref[...] = jnp.dot(xt, tht_ref[...], preferred_element_type=jnp.float32)
    yi_ref[...] = jnp.dot(xi, thi_ref[...], preferred_element_type=jnp.float32)


def _layer_matmul(xt, tht, xi, thi, dinv=None, bt=None, bi=None):
    xspec = pl.BlockSpec((_MMBLK, EMB), lambda i: (i, 0))
    tspec = pl.BlockSpec((EMB, EMB), lambda i: (0, 0))
    bspec = pl.BlockSpec((1, EMB), lambda i: (0, 0))
    dspec = pl.BlockSpec((_MMBLK, 1), lambda i: (i, 0))
    out_shape = [jax.ShapeDtypeStruct((N_NODES, EMB), jnp.float32)] * 2
    if dinv is None:
        return pl.pallas_call(
            _mm2_body,
            grid=(N_NODES // _MMBLK,),
            in_specs=[xspec, tspec, xspec, tspec],
            out_specs=[xspec, xspec],
            out_shape=out_shape,
        )(xt, tht, xi, thi)
    return pl.pallas_call(
        _mm2_scaled_body,
        grid=(N_NODES // _MMBLK,),
        in_specs=[xspec, tspec, xspec, tspec, dspec, bspec, bspec],
        out_specs=[xspec, xspec],
        out_shape=out_shape,
    )(xt, tht, xi, thi, dinv.reshape(N_NODES, 1),
      bt.reshape(1, EMB), bi.reshape(1, EMB))


def _escale_body(et_ref, ei_ref, b_ref, ot_ref, oi_ref):
    b = b_ref[...]
    ot_ref[...] = et_ref[...] * b
    oi_ref[...] = ei_ref[...] * b


def _escale(et, ei, binv):
    xspec = pl.BlockSpec((_MMBLK, EMB), lambda i: (i, 0))
    dspec = pl.BlockSpec((_MMBLK, 1), lambda i: (i, 0))
    return pl.pallas_call(
        _escale_body,
        grid=(N_NODES // _MMBLK,),
        in_specs=[xspec, xspec, dspec],
        out_specs=[xspec, xspec],
        out_shape=[jax.ShapeDtypeStruct((N_NODES, EMB), jnp.float32)] * 2,
    )(et, ei, binv.reshape(N_NODES, 1))


_NCHT = E_INC // ICH


def _ilv_body(n_ref, e_ref, o_ref):
    o_ref[:, 0, :] = n_ref[...]
    o_ref[:, 1, :] = e_ref[...]


def _interleave_idx(node, edge):
    blk = 400
    spec = pl.BlockSpec((blk, ICH), lambda i: (i, 0))
    out = pl.pallas_call(
        _ilv_body,
        grid=(_NCHT // blk,),
        in_specs=[spec, spec],
        out_specs=pl.BlockSpec((blk, 2, ICH), lambda i: (i, 0, 0)),
        out_shape=jax.ShapeDtypeStruct((_NCHT, 2, ICH), jnp.int32),
    )(node.reshape(_NCHT, ICH), edge.reshape(_NCHT, ICH))
    return out.reshape(2 * E_INC)


def _head_body(tg0_ref, ig0_ref, d0_ref, bt1_ref, bi1_ref,
               user_ref, w3_ref, b3_ref,
               wq_ref, bq_ref, wk_ref, bk_ref, wv_ref, bv_ref,
               wo_ref, bo_ref, wf1_ref, bf1_ref, wf2_ref, bf2_ref,
               ln1g_ref, ln1b_ref, ln2g_ref, ln2b_ref,
               dw_ref, db_ref, w4_ref, b4_ref, out_ref):
    f32 = jnp.float32

    def mm(a, b):
        return jnp.dot(a, b, preferred_element_type=f32)

    def layer_norm(x, g, b):
        m = jnp.mean(x, axis=-1, keepdims=True)
        v = jnp.mean((x - m) * (x - m), axis=-1, keepdims=True)
        return (x - m) / jnp.sqrt(v + 1e-5) * g + b

    d0 = d0_ref[...]
    tg0 = tg0_ref[...] * d0 + bt1_ref[...]
    ig0 = ig0_ref[...] * d0 + bi1_ref[...]
    user = user_ref[...][:, :HALF]
    text_user = jnp.concatenate([tg0, user], axis=1)
    img_user = jnp.concatenate([ig0, user], axis=1)
    tiu = mm(jnp.concatenate([text_user, img_user], axis=1),
             w3_ref[...]) + b3_ref[...]
    q = mm(tiu, wq_ref[...]) + bq_ref[...]
    k1 = mm(text_user, wk_ref[...]) + bk_ref[...]
    k2 = mm(img_user, wk_ref[...]) + bk_ref[...]
    v1 = mm(text_user, wv_ref[...]) + bv_ref[...]
    v2 = mm(img_user, wv_ref[...]) + bv_ref[...]
    scale = np.float32(1.0 / np.sqrt(DH))
    ao_parts = []
    for h in range(HEADS):
        sl = slice(h * DH, (h + 1) * DH)
        qh, k1h, k2h = q[:, sl], k1[:, sl], k2[:, sl]
        s1 = jnp.sum(qh * k1h, axis=1, keepdims=True) * scale
        s2 = jnp.sum(qh * k2h, axis=1, keepdims=True) * scale
        m = jnp.maximum(s1, s2)
        e1 = jnp.exp(s1 - m)
        e2 = jnp.exp(s2 - m)
        tot = e1 + e2
        ao_parts.append((e1 / tot) * v1[:, sl] + (e2 / tot) * v2[:, sl])
    ao = mm(jnp.concatenate(ao_parts, axis=1), wo_ref[...]) + bo_ref[...]
    x = layer_norm(tiu + ao, ln1g_ref[...], ln1b_ref[...])
    ff = mm(jnp.maximum(mm(x, wf1_ref[...]) + bf1_ref[...], 0.0),
            wf2_ref[...]) + bf2_ref[...]
    x = layer_norm(x + ff, ln2g_ref[...], ln2b_ref[...])
    x = jnp.maximum(mm(x, dw_ref[...]) + db_ref[...], 0.0)
    out_ref[...] = mm(x, w4_ref[...]) + b4_ref[...]


def _head(tg0, ig0, d0, bt1, bi1, user, p):
    w4p = jnp.pad(p['W4'], ((0, 0), (0, 127)))
    b4p = jnp.pad(p['b4'], (0, 127)).reshape(1, 128)
    args = [tg0, ig0, d0, bt1.reshape(1, EMB), bi1.reshape(1, EMB), user,
            p['W3'], p['b3'].reshape(1, D),
            p['Wq'], p['bq'].reshape(1, D), p['Wk'], p['bk'].reshape(1, D),
            p['Wv'], p['bv'].reshape(1, D), p['Wo'], p['bo'].reshape(1, D),
            p['Wf1'], p['bf1'].reshape(1, D), p['Wf2'], p['bf2'].reshape(1, D),
            p['ln1_g'].reshape(1, D), p['ln1_b'].reshape(1, D),
            p['ln2_g'].reshape(1, D), p['ln2_b'].reshape(1, D),
            p['dW'], p['db'].reshape(1, D), w4p, b4p]
    out = pl.pallas_call(
        _head_body,
        out_shape=jax.ShapeDtypeStruct((BSZ, 128), jnp.float32),
    )(*args)
    return out[:, :1]


_MESH = plsc.VectorSubcoreMesh(
    core_axis_name="c", subcore_axis_name="s", num_cores=NC, num_subcores=NS)


@functools.partial(
    pl.kernel,
    out_type=[
        jax.ShapeDtypeStruct((N_NODES, EMB), jnp.float32),
        jax.ShapeDtypeStruct((N_NODES, EMB), jnp.float32),
        jax.ShapeDtypeStruct((BSZ, EMB), jnp.float32),
        jax.ShapeDtypeStruct((N_NODES,), jnp.float32),
        jax.ShapeDtypeStruct((N_NODES,), jnp.float32),
    ],
    mesh=_MESH,
    scratch_types=[
        pltpu.VMEM((XCH,), jnp.int32),
        pltpu.VMEM((XCH, EMB), jnp.float32),
        pltpu.VMEM((LENS, EMB), jnp.float32),
        pltpu.VMEM_SHARED((N_NODES,), jnp.float32),
        pltpu.VMEM((RWT * 2,), jnp.float32),
        pltpu.VMEM((ICH,), jnp.float32),
        pltpu.VMEM((ICH,), jnp.int32),
        pltpu.VMEM((8,), jnp.int32),
        pltpu.VMEM((8, EMB), jnp.float32),
        pltpu.SemaphoreType.DMA,
    ],
)
def _sc_prep(pt_hbm, pi_hbm, ri_hbm, node_hbm, edge_hbm, uid_hbm, ut_hbm,
             pos_hbm, xt_hbm, xi_hbm, user_hbm, dinv_hbm, binv_hbm,
             idxv, rows, posv, deg, degv, onesv, div, uidv, urows, sem):
    c = lax.axis_index("c")
    s = lax.axis_index("s")
    w = c * NS + s
    nb = w * (N_NODES // (NC * NS))
    dpt = RWT * 2

    def zdeg(j, _):
        degv[pl.ds(j * 16, 16)] = jnp.zeros((16,), jnp.float32)
        return 0
    lax.fori_loop(0, dpt // 16, zdeg, 0)
    pltpu.sync_copy(degv, deg.at[pl.ds(s * dpt, dpt)])
    for j in range(ICH // 16):
        onesv[pl.ds(j * 16, 16)] = jnp.ones((16,), jnp.float32)
    plsc.subcore_barrier()

    def deg_pass(src_hbm):
        def body(ch, _):
            base = s * EPT + ch * ICH
            pltpu.sync_copy(src_hbm.at[pl.ds(base, ICH)], div)
            pltpu.sync_copy(onesv, deg.at[div], add=True)
            return 0
        lax.fori_loop(0, NCH, body, 0)

    @pl.when(c == 0)
    def _():
        deg_pass(node_hbm)

    @pl.when(c == 1)
    def _():
        deg_pass(edge_hbm)

    pltpu.sync_copy(pos_hbm, posv)
    for p_hbm, x_hbm in ((pt_hbm, xt_hbm), (pi_hbm, xi_hbm)):
        def xbody(k, _):
            pltpu.sync_copy(ri_hbm.at[pl.ds(nb + k * XCH, XCH)], idxv)
            pltpu.async_copy(p_hbm.at[idxv], rows, sem).wait()

            def posadd(r, _):
                pr = lax.rem(k * XCH + r, LENS)
                for j in range(EMB // 16):
                    sl = pl.ds(j * 16, 16)
                    rows[r, sl] = rows[r, sl] + posv[pr, sl]
                return 0
            lax.fori_loop(0, XCH, posadd, 0)
            pltpu.sync_copy(rows, x_hbm.at[pl.ds(nb + k * XCH, XCH)])
            return 0
        lax.fori_loop(0, (N_NODES // (NC * NS)) // XCH, xbody, 0)

    @pl.when((c == 0) & (s < 8))
    def _():
        pltpu.sync_copy(uid_hbm.at[pl.ds(s * 8, 8)], uidv)
        pltpu.async_copy(ut_hbm.at[uidv], urows, sem).wait()
        pltpu.sync_copy(urows, user_hbm.at[pl.ds(s * 8, 8)])

    plsc.subcore_barrier()

    pltpu.sync_copy(deg.at[pl.ds(s * dpt, dpt)], degv)

    def inv(j, _):
        sl = pl.ds(j * 16, 16)
        v = degv[sl]
        degv[sl] = jnp.where(v > 0.0, 1.0 / v, 0.0)
        return 0
    lax.fori_loop(0, dpt // 16, inv, 0)

    @pl.when(c == 0)
    def _():
        pltpu.sync_copy(degv, dinv_hbm.at[pl.ds(s * dpt, dpt)])

    @pl.when(c == 1)
    def _():
        pltpu.sync_copy(degv, binv_hbm.at[pl.ds(s * dpt, dpt)])


def _make_seg(src_off, dst_off):
    @functools.partial(
        pl.kernel,
        out_type=[
            jax.ShapeDtypeStruct((N_NODES, EMB), jnp.float32),
            jax.ShapeDtypeStruct((N_NODES, EMB), jnp.float32),
        ],
        mesh=_MESH,
        scratch_types=[
            pltpu.VMEM_SHARED((RNG + NDUM, EMB), jnp.float32),
            pltpu.VMEM((2 * ICH,), jnp.int32),
            pltpu.VMEM((2 * ICH,), jnp.int32),
            pltpu.VMEM((ICH,), jnp.int32),
            pltpu.VMEM((ICH,), jnp.int32),
            pltpu.VMEM((ICH,), jnp.int32),
            pltpu.VMEM((ICH,), jnp.int32),
            pltpu.VMEM((ICH, EMB), jnp.float32),
            pltpu.VMEM((ICH, EMB), jnp.float32),
            pltpu.VMEM((40, EMB), jnp.float32),
            pltpu.SemaphoreType.DMA,
            pltpu.SemaphoreType.DMA,
            pltpu.SemaphoreType.DMA,
            pltpu.SemaphoreType.DMA,
        ],
    )
    def seg(gt_hbm, gi_hbm, idx2_hbm, ot_hbm, oi_hbm,
            acc, ibA, ibB, nvA, nvB, evcA, evcB, rowsA, rowsB, zbuf,
            gsA, gsB, ssA, ssB):
        c = lax.axis_index("c")
        s = lax.axis_index("s")
        lo = c * RNG
        r0 = s * RWT
        bufs = ((ibA, nvA, evcA, rowsA, gsA, ssA),
                (ibB, nvB, evcB, rowsB, gsB, ssB))

        def zb(r, _):
            for j in range(EMB // 16):
                zbuf[r, pl.ds(j * 16, 16)] = jnp.zeros((16,), jnp.float32)
            return 0
        lax.fori_loop(0, 40, zb, 0)

        def load_idx_and_gather(cx, ib, nv, rows, gs, g_hbm):
            pltpu.sync_copy(
                idx2_hbm.at[pl.ds((s * NCH + cx) * 2 * ICH, 2 * ICH)], ib)
            for j in range(ICH // 16):
                nv[pl.ds(j * 16, 16)] = ib[pl.ds(src_off + j * 16, 16)]
            pltpu.async_copy(g_hbm.at[nv], rows, gs)

        for m, (g_hbm, o_hbm) in enumerate(((gt_hbm, ot_hbm),
                                            (gi_hbm, oi_hbm))):
            for k in range(RWT // 40):
                pltpu.sync_copy(zbuf, acc.at[pl.ds(r0 + k * 40, 40)])
            plsc.subcore_barrier()

            load_idx_and_gather(0, ibA, nvA, rowsA, gsA, g_hbm)
            load_idx_and_gather(1, ibB, nvB, rowsB, gsB, g_hbm)

            def body(q, _):
                for x, (ib, nv, evc, rows, gs, ss) in enumerate(bufs):
                    cx = 2 * q + x
                    pltpu.make_async_copy(g_hbm.at[nv], rows, gs).wait()
                    for j in range(ICH // 16):
                        sl = pl.ds(j * 16, 16)
                        e = ib[pl.ds(dst_off + j * 16, 16)]
                        loc = e - lo
                        dummy = RNG + (e & (NDUM - 1))
                        ok = (loc >= 0) & (loc < RNG)
                        evc[sl] = jnp.where(ok, loc, dummy)
                    pltpu.async_copy(rows, acc.at[evc], ss, add=True)
                    pltpu.make_async_copy(rows, acc.at[evc], ss).wait()

                    @pl.when(cx + 2 < NCH)
                    def _():
                        load_idx_and_gather(cx + 2, ib, nv, rows, gs, g_hbm)
                return 0
            lax.fori_loop(0, NCH // 2, body, 0)
            plsc.subcore_barrier()

            pltpu.sync_copy(acc.at[pl.ds(r0, RWT)],
                            o_hbm.at[pl.ds(lo + r0, RWT)])
            if m == 0:
                plsc.subcore_barrier()

    return seg


_seg_s1 = _make_seg(0, ICH)
_seg_s2 = _make_seg(ICH, 0)


def kernel(input, hg_idx, related_items, label, uid, params):
    p = params
    node = hg_idx[0]
    edge = hg_idx[1]

    pt = _project_table(p['text_table'], p['W1'], p['b1'], 400)
    pi = _project_table(p['img_table'], p['W2'], p['b2'], 400)

    pos = jnp.asarray(_POS)
    ut_p = jnp.pad(p['user_table'], ((0, 0), (0, EMB - HALF)))
    xt, xi, user, dinv, binv = _sc_prep(
        pt, pi, related_items, node, edge, uid, ut_p, pos)

    idx2 = _interleave_idx(node, edge)

    yt, yi = _layer_matmul(xt, p['theta_t0'], xi, p['theta_i0'])
    et_raw, ei_raw = _seg_s1(yt, yi, idx2)
    et, ei = _escale(et_raw, ei_raw, binv)
    xt2, xi2 = _seg_s2(et, ei, idx2)

    yt2, yi2 = _layer_matmul(xt2, p['theta_t1'], xi2, p['theta_i1'],
                             dinv, p['bias_t0'], p['bias_i0'])
    et2_raw, ei2_raw = _seg_s1(yt2, yi2, idx2)
    et2, ei2 = _escale(et2_raw, ei2_raw, binv)
    xt3, xi3 = _seg_s2(et2, ei2, idx2)
    tg0_raw = xt3[::LENS]
    ig0_raw = xi3[::LENS]

    d0 = dinv[::LENS].reshape(BSZ, 1)
    return _head(tg0_raw, ig0_raw, d0, p['bias_t1'], p['bias_i1'], user, p)


# speedup vs baseline: 8.0363x; 1.4492x over previous
import functools

import jax
import jax.numpy as jnp
import numpy as np
from jax import lax
from jax.experimental import pallas as pl
from jax.experimental.pallas import tpu as pltpu
from jax.experimental.pallas import tpu_sc as plsc

BSZ = 64
LENS = 300
EMB = 128
HALF = EMB // 2
D = EMB + HALF
N_NODES = BSZ * LENS
E_INC = 307200
N_ITEMS = 20000
HEADS = 4
DH = D // HEADS

NC, NS = 2, 16
EPT = E_INC // NS
ICH = 128
NCH = EPT // ICH
RNG = N_NODES // NC
NDUM = 512
RWT = RNG // NS
XCH = 120
BPT = RWT // LENS


def _make_pos():
    position = np.arange(LENS)[:, None].astype(np.float64)
    div_term = np.exp(np.arange(0, EMB, 2) * (-np.log(10000.0) / EMB))
    pe = np.zeros((LENS + 1, EMB))
    pe[1:, 0::2] = np.sin(position * div_term)
    pe[1:, 1::2] = np.cos(position * div_term)
    return np.ascontiguousarray((pe[:LENS] * 0.002).astype(np.float32))


_POS = _make_pos()


def _proj_body(tab_ref, w_ref, b_ref, o_ref):
    o_ref[...] = jnp.dot(tab_ref[...], w_ref[...],
                         preferred_element_type=jnp.float32) + b_ref[...]


def _project_table(table, w, b, rows_blk):
    n, k = table.shape
    return pl.pallas_call(
        _proj_body,
        grid=(n // rows_blk,),
        in_specs=[
            pl.BlockSpec((rows_blk, k), lambda i: (i, 0)),
            pl.BlockSpec((k, EMB), lambda i: (0, 0)),
            pl.BlockSpec((1, EMB), lambda i: (0, 0)),
        ],
        out_specs=pl.BlockSpec((rows_blk, EMB), lambda i: (i, 0)),
        out_shape=jax.ShapeDtypeStruct((n, EMB), jnp.float32),
    )(table, w, b.reshape(1, EMB))


_MMBLK = 1200


def _mm2_body(xt_ref, tht_ref, xi_ref, thi_ref, yt_ref, yi_ref):
    yt_ref[...] = jnp.dot(xt_ref[...], tht_ref[...],
                          preferred_element_type=jnp.float32)
    yi_ref[...] = jnp.dot(xi_ref[...], thi_ref[...],
                          preferred_element_type=jnp.float32)


def _mm2_scaled_body(xt_ref, tht_ref, xi_ref, thi_ref, d_ref,
                     bt_ref, bi_ref, yt_ref, yi_ref):
    d = d_ref[...]
    xt = xt_ref[...] * d + bt_ref[...]
    xi = xi_ref[...] * d + bi_ref[...]
    yt_ref[...] = jnp.dot(xt, tht_ref[...], preferred_element_type=jnp.float32)
    yi_ref[...] = jnp.dot(xi, thi_ref[...], preferred_element_type=jnp.float32)


def _layer_matmul(xt, tht, xi, thi, dinv=None, bt=None, bi=None):
    xspec = pl.BlockSpec((_MMBLK, EMB), lambda i: (i, 0))
    tspec = pl.BlockSpec((EMB, EMB), lambda i: (0, 0))
    bspec = pl.BlockSpec((1, EMB), lambda i: (0, 0))
    dspec = pl.BlockSpec((_MMBLK, 1), lambda i: (i, 0))
    out_shape = [jax.ShapeDtypeStruct((N_NODES, EMB), jnp.float32)] * 2
    if dinv is None:
        return pl.pallas_call(
            _mm2_body,
            grid=(N_NODES // _MMBLK,),
            in_specs=[xspec, tspec, xspec, tspec],
            out_specs=[xspec, xspec],
            out_shape=out_shape,
        )(xt, tht, xi, thi)
    return pl.pallas_call(
        _mm2_scaled_body,
        grid=(N_NODES // _MMBLK,),
        in_specs=[xspec, tspec, xspec, tspec, dspec, bspec, bspec],
        out_specs=[xspec, xspec],
        out_shape=out_shape,
    )(xt, tht, xi, thi, dinv.reshape(N_NODES, 1),
      bt.reshape(1, EMB), bi.reshape(1, EMB))


def _escale_body(et_ref, ei_ref, b_ref, ot_ref, oi_ref):
    b = b_ref[...]
    ot_ref[...] = et_ref[...] * b
    oi_ref[...] = ei_ref[...] * b


def _escale(et, ei, binv):
    xspec = pl.BlockSpec((_MMBLK, EMB), lambda i: (i, 0))
    dspec = pl.BlockSpec((_MMBLK, 1), lambda i: (i, 0))
    return pl.pallas_call(
        _escale_body,
        grid=(N_NODES // _MMBLK,),
        in_specs=[xspec, xspec, dspec],
        out_specs=[xspec, xspec],
        out_shape=[jax.ShapeDtypeStruct((N_NODES, EMB), jnp.float32)] * 2,
    )(et, ei, binv.reshape(N_NODES, 1))


def _head_body(tg0_ref, ig0_ref, d0_ref, bt1_ref, bi1_ref,
               user_ref, w3_ref, b3_ref,
               wq_ref, bq_ref, wk_ref, bk_ref, wv_ref, bv_ref,
               wo_ref, bo_ref, wf1_ref, bf1_ref, wf2_ref, bf2_ref,
               ln1g_ref, ln1b_ref, ln2g_ref, ln2b_ref,
               dw_ref, db_ref, w4_ref, b4_ref, out_ref):
    f32 = jnp.float32

    def mm(a, b):
        return jnp.dot(a, b, preferred_element_type=f32)

    def layer_norm(x, g, b):
        m = jnp.mean(x, axis=-1, keepdims=True)
        v = jnp.mean((x - m) * (x - m), axis=-1, keepdims=True)
        return (x - m) / jnp.sqrt(v + 1e-5) * g + b

    d0 = d0_ref[...]
    tg0 = tg0_ref[...] * d0 + bt1_ref[...]
    ig0 = ig0_ref[...] * d0 + bi1_ref[...]
    user = user_ref[...][:, :HALF]
    text_user = jnp.concatenate([tg0, user], axis=1)
    img_user = jnp.concatenate([ig0, user], axis=1)
    tiu = mm(jnp.concatenate([text_user, img_user], axis=1),
             w3_ref[...]) + b3_ref[...]
    q = mm(tiu, wq_ref[...]) + bq_ref[...]
    k1 = mm(text_user, wk_ref[...]) + bk_ref[...]
    k2 = mm(img_user, wk_ref[...]) + bk_ref[...]
    v1 = mm(text_user, wv_ref[...]) + bv_ref[...]
    v2 = mm(img_user, wv_ref[...]) + bv_ref[...]
    scale = np.float32(1.0 / np.sqrt(DH))
    ao_parts = []
    for h in range(HEADS):
        sl = slice(h * DH, (h + 1) * DH)
        qh, k1h, k2h = q[:, sl], k1[:, sl], k2[:, sl]
        s1 = jnp.sum(qh * k1h, axis=1, keepdims=True) * scale
        s2 = jnp.sum(qh * k2h, axis=1, keepdims=True) * scale
        m = jnp.maximum(s1, s2)
        e1 = jnp.exp(s1 - m)
        e2 = jnp.exp(s2 - m)
        tot = e1 + e2
        ao_parts.append((e1 / tot) * v1[:, sl] + (e2 / tot) * v2[:, sl])
    ao = mm(jnp.concatenate(ao_parts, axis=1), wo_ref[...]) + bo_ref[...]
    x = layer_norm(tiu + ao, ln1g_ref[...], ln1b_ref[...])
    ff = mm(jnp.maximum(mm(x, wf1_ref[...]) + bf1_ref[...], 0.0),
            wf2_ref[...]) + bf2_ref[...]
    x = layer_norm(x + ff, ln2g_ref[...], ln2b_ref[...])
    x = jnp.maximum(mm(x, dw_ref[...]) + db_ref[...], 0.0)
    out_ref[...] = mm(x, w4_ref[...]) + b4_ref[...]


def _head(tg0, ig0, d0, bt1, bi1, user, p):
    w4p = jnp.pad(p['W4'], ((0, 0), (0, 127)))
    b4p = jnp.pad(p['b4'], (0, 127)).reshape(1, 128)
    args = [tg0, ig0, d0, bt1.reshape(1, EMB), bi1.reshape(1, EMB), user,
            p['W3'], p['b3'].reshape(1, D),
            p['Wq'], p['bq'].reshape(1, D), p['Wk'], p['bk'].reshape(1, D),
            p['Wv'], p['bv'].reshape(1, D), p['Wo'], p['bo'].reshape(1, D),
            p['Wf1'], p['bf1'].reshape(1, D), p['Wf2'], p['bf2'].reshape(1, D),
            p['ln1_g'].reshape(1, D), p['ln1_b'].reshape(1, D),
            p['ln2_g'].reshape(1, D), p['ln2_b'].reshape(1, D),
            p['dW'], p['db'].reshape(1, D), w4p, b4p]
    out = pl.pallas_call(
        _head_body,
        out_shape=jax.ShapeDtypeStruct((BSZ, 128), jnp.float32),
    )(*args)
    return out[:, :1]


_MESH = plsc.VectorSubcoreMesh(
    core_axis_name="c", subcore_axis_name="s", num_cores=NC, num_subcores=NS)


_PADK = 0x7FFF


@functools.partial(
    pl.kernel,
    out_type=[
        jax.ShapeDtypeStruct((N_NODES, EMB), jnp.float32),
        jax.ShapeDtypeStruct((N_NODES, EMB), jnp.float32),
        jax.ShapeDtypeStruct((BSZ, EMB), jnp.float32),
        jax.ShapeDtypeStruct((N_NODES,), jnp.float32),
        jax.ShapeDtypeStruct((N_NODES,), jnp.float32),
        jax.ShapeDtypeStruct((NC, NS, EPT), jnp.int32),
        jax.ShapeDtypeStruct((NC, NS, 16), jnp.int32),
        jax.ShapeDtypeStruct((NC, NS, EPT), jnp.int32),
        jax.ShapeDtypeStruct((NC, NS, 16), jnp.int32),
    ],
    mesh=_MESH,
    compiler_params=pltpu.CompilerParams(needs_layout_passes=False),
    scratch_types=[
        pltpu.VMEM((XCH,), jnp.int32),
        pltpu.VMEM((XCH, EMB), jnp.float32),
        pltpu.VMEM((LENS, EMB), jnp.float32),
        pltpu.VMEM_SHARED((N_NODES,), jnp.float32),
        pltpu.VMEM((RWT * 2,), jnp.float32),
        pltpu.VMEM((ICH,), jnp.float32),
        pltpu.VMEM((ICH,), jnp.int32),
        pltpu.VMEM((ICH,), jnp.int32),
        pltpu.VMEM((EPT,), jnp.int32),
        pltpu.VMEM((EPT,), jnp.int32),
        pltpu.VMEM((16,), jnp.int32),
        pltpu.VMEM((8,), jnp.int32),
        pltpu.VMEM((8, EMB), jnp.float32),
        pltpu.SemaphoreType.DMA,
    ],
)
def _sc_prep(pt_hbm, pi_hbm, ri_hbm, node_hbm, edge_hbm, uid_hbm, ut_hbm,
             pos_hbm, xt_hbm, xi_hbm, user_hbm, dinv_hbm, binv_hbm,
             pke_hbm, cnte_hbm, pkn_hbm, cntn_hbm,
             idxv, rows, posv, deg, degv, onesv, nbv, ebv, pebuf, pnbuf,
             cntv, uidv, urows, sem):
    c = lax.axis_index("c")
    s = lax.axis_index("s")
    w = c * NS + s
    nb = w * (N_NODES // (NC * NS))
    dpt = RWT * 2
    lo = c * RNG
    hi = lo + RNG

    def zdeg(j, _):
        degv[pl.ds(j * 16, 16)] = jnp.zeros((16,), jnp.float32)
        return 0
    lax.fori_loop(0, dpt // 16, zdeg, 0)
    pltpu.sync_copy(degv, deg.at[pl.ds(s * dpt, dpt)])
    for j in range(ICH // 16):
        onesv[pl.ds(j * 16, 16)] = jnp.ones((16,), jnp.float32)

    lane = lax.iota(jnp.int32, 16)
    pad_e = lane | (_PADK << 15)
    pad_n = _PADK | (lane << 15)

    def zpk(j, _):
        sl = pl.ds(j * 16, 16)
        pebuf[sl] = pad_e
        pnbuf[sl] = pad_n
        return 0
    lax.fori_loop(0, EPT // 16, zpk, 0)
    plsc.subcore_barrier()

    def part_body(ch, carry):
        ce, cn = carry
        base = s * EPT + ch * ICH
        pltpu.sync_copy(node_hbm.at[pl.ds(base, ICH)], nbv)
        pltpu.sync_copy(edge_hbm.at[pl.ds(base, ICH)], ebv)

        @pl.when(c == 0)
        def _():
            pltpu.sync_copy(onesv, deg.at[nbv], add=True)

        @pl.when(c == 1)
        def _():
            pltpu.sync_copy(onesv, deg.at[ebv], add=True)

        one = jnp.ones((16,), jnp.int32)
        zero = jnp.zeros((16,), jnp.int32)
        for j in range(ICH // 16):
            sl = pl.ds(j * 16, 16)
            nv = nbv[sl]
            ev = ebv[sl]
            pk = nv | (ev << 15)
            me = (ev >= lo) & (ev < hi)
            cse = plsc.cumsum(jnp.where(me, one, zero))
            plsc.store_scatter(pebuf, [ce + cse - 1], pk, mask=me)
            ce = ce + jnp.max(cse)
            mn = (nv >= lo) & (nv < hi)
            csn = plsc.cumsum(jnp.where(mn, one, zero))
            plsc.store_scatter(pnbuf, [cn + csn - 1], pk, mask=mn)
            cn = cn + jnp.max(csn)
        return ce, cn

    ce, cn = lax.fori_loop(0, NCH, part_body, (jnp.int32(0), jnp.int32(0)))

    qe = jnp.maximum(jnp.minimum((ce + 255) // 256 * 256, EPT) // 256, 1)
    qn = jnp.maximum(jnp.minimum((cn + 255) // 256 * 256, EPT) // 256, 1)
    cntv[pl.ds(0, 16)] = jnp.full((16,), qe, jnp.int32)
    pltpu.sync_copy(cntv, cnte_hbm.at[c, s])
    pltpu.sync_copy(pebuf, pke_hbm.at[c, s])
    cntv[pl.ds(0, 16)] = jnp.full((16,), qn, jnp.int32)
    pltpu.sync_copy(cntv, cntn_hbm.at[c, s])
    pltpu.sync_copy(pnbuf, pkn_hbm.at[c, s])

    pltpu.sync_copy(pos_hbm, posv)
    for p_hbm, x_hbm in ((pt_hbm, xt_hbm), (pi_hbm, xi_hbm)):
        def xbody(k, _):
            pltpu.sync_copy(ri_hbm.at[pl.ds(nb + k * XCH, XCH)], idxv)
            pltpu.async_copy(p_hbm.at[idxv], rows, sem).wait()

            def posadd(r, _):
                pr = lax.rem(k * XCH + r, LENS)
                for j in range(EMB // 16):
                    sl = pl.ds(j * 16, 16)
                    rows[r, sl] = rows[r, sl] + posv[pr, sl]
                return 0
            lax.fori_loop(0, XCH, posadd, 0)
            pltpu.sync_copy(rows, x_hbm.at[pl.ds(nb + k * XCH, XCH)])
            return 0
        lax.fori_loop(0, (N_NODES // (NC * NS)) // XCH, xbody, 0)

    @pl.when((c == 0) & (s < 8))
    def _():
        pltpu.sync_copy(uid_hbm.at[pl.ds(s * 8, 8)], uidv)
        pltpu.async_copy(ut_hbm.at[uidv], urows, sem).wait()
        pltpu.sync_copy(urows, user_hbm.at[pl.ds(s * 8, 8)])

    plsc.subcore_barrier()

    pltpu.sync_copy(deg.at[pl.ds(s * dpt, dpt)], degv)

    def inv(j, _):
        sl = pl.ds(j * 16, 16)
        v = degv[sl]
        degv[sl] = jnp.where(v > 0.0, 1.0 / v, 0.0)
        return 0
    lax.fori_loop(0, dpt // 16, inv, 0)

    @pl.when(c == 0)
    def _():
        pltpu.sync_copy(degv, dinv_hbm.at[pl.ds(s * dpt, dpt)])

    @pl.when(c == 1)
    def _():
        pltpu.sync_copy(degv, binv_hbm.at[pl.ds(s * dpt, dpt)])


def _make_seg(key_low):
    @functools.partial(
        pl.kernel,
        out_type=[
            jax.ShapeDtypeStruct((N_NODES, EMB), jnp.float32),
            jax.ShapeDtypeStruct((N_NODES, EMB), jnp.float32),
        ],
        mesh=_MESH,
        compiler_params=pltpu.CompilerParams(needs_layout_passes=False),
        scratch_types=[
            pltpu.VMEM_SHARED((RNG + NDUM, EMB), jnp.float32),
            pltpu.VMEM((ICH,), jnp.int32),
            pltpu.VMEM((ICH,), jnp.int32),
            pltpu.VMEM((ICH,), jnp.int32),
            pltpu.VMEM((ICH,), jnp.int32),
            pltpu.VMEM((ICH,), jnp.int32),
            pltpu.VMEM((ICH,), jnp.int32),
            pltpu.VMEM((ICH, EMB), jnp.float32),
            pltpu.VMEM((ICH, EMB), jnp.float32),
            pltpu.VMEM((40, EMB), jnp.float32),
            pltpu.VMEM((16,), jnp.int32),
            pltpu.SemaphoreType.DMA,
            pltpu.SemaphoreType.DMA,
            pltpu.SemaphoreType.DMA,
            pltpu.SemaphoreType.DMA,
        ],
    )
    def seg(gt_hbm, gi_hbm, pk_hbm, cnt_hbm, ot_hbm, oi_hbm,
            acc, pkA, pkB, nvA, nvB, evcA, evcB, rowsA, rowsB, zbuf, cbuf,
            gsA, gsB, ssA, ssB):
        c = lax.axis_index("c")
        s = lax.axis_index("s")
        lo = c * RNG
        r0 = s * RWT
        bufs = ((pkA, nvA, evcA, rowsA, gsA, ssA),
                (pkB, nvB, evcB, rowsB, gsB, ssB))

        def zb(r, _):
            for j in range(EMB // 16):
                zbuf[r, pl.ds(j * 16, 16)] = jnp.zeros((16,), jnp.float32)
            return 0
        lax.fori_loop(0, 40, zb, 0)

        pltpu.sync_copy(cnt_hbm.at[c, s], cbuf)
        qtrips = jnp.max(cbuf[pl.ds(0, 16)])

        def decode(pk16):
            if key_low:
                return pk16 >> 15, pk16 & _PADK
            return pk16 & _PADK, pk16 >> 15

        def load_idx_and_gather(cx, pk, nv, rows, gs, g_hbm):
            pltpu.sync_copy(pk_hbm.at[c, s, pl.ds(cx * ICH, ICH)], pk)
            for j in range(ICH // 16):
                sl = pl.ds(j * 16, 16)
                g, _ = decode(pk[sl])
                nv[sl] = g
            pltpu.async_copy(g_hbm.at[nv], rows, gs)

        for m, (g_hbm, o_hbm) in enumerate(((gt_hbm, ot_hbm),
                                            (gi_hbm, oi_hbm))):
            for k in range(RWT // 40):
                pltpu.sync_copy(zbuf, acc.at[pl.ds(r0 + k * 40, 40)])
            plsc.subcore_barrier()

            load_idx_and_gather(0, pkA, nvA, rowsA, gsA, g_hbm)
            load_idx_and_gather(1, pkB, nvB, rowsB, gsB, g_hbm)

            def body(q, _):
                for x, (pk, nv, evc, rows, gs, ss) in enumerate(bufs):
                    cx = 2 * q + x
                    pltpu.make_async_copy(g_hbm.at[nv], rows, gs).wait()
                    for j in range(ICH // 16):
                        sl = pl.ds(j * 16, 16)
                        _, e = decode(pk[sl])
                        loc = e - lo
                        dummy = RNG + (e & (NDUM - 1))
                        ok = (loc >= 0) & (loc < RNG)
                        evc[sl] = jnp.where(ok, loc, dummy)
                    pltpu.async_copy(rows, acc.at[evc], ss, add=True)
                    pltpu.make_async_copy(rows, acc.at[evc], ss).wait()

                    @pl.when(cx + 2 < 2 * qtrips)
                    def _():
                        load_idx_and_gather(cx + 2, pk, nv, rows, gs, g_hbm)
                return 0
            lax.fori_loop(0, qtrips, body, 0)
            plsc.subcore_barrier()

            pltpu.sync_copy(acc.at[pl.ds(r0, RWT)],
                            o_hbm.at[pl.ds(lo + r0, RWT)])
            if m == 0:
                plsc.subcore_barrier()

    return seg


_seg_s1 = _make_seg(False)
_seg_s2 = _make_seg(True)


def kernel(input, hg_idx, related_items, label, uid, params):
    p = params
    node = hg_idx[0]
    edge = hg_idx[1]

    pt = _project_table(p['text_table'], p['W1'], p['b1'], 400)
    pi = _project_table(p['img_table'], p['W2'], p['b2'], 400)

    pos = jnp.asarray(_POS)
    ut_p = jnp.pad(p['user_table'], ((0, 0), (0, EMB - HALF)))
    (xt, xi, user, dinv, binv,
     pk_e, cnt_e, pk_n, cnt_n) = _sc_prep(
        pt, pi, related_items, node, edge, uid, ut_p, pos)

    yt, yi = _layer_matmul(xt, p['theta_t0'], xi, p['theta_i0'])
    et_raw, ei_raw = _seg_s1(yt, yi, pk_e, cnt_e)
    et, ei = _escale(et_raw, ei_raw, binv)
    xt2, xi2 = _seg_s2(et, ei, pk_n, cnt_n)

    yt2, yi2 = _layer_matmul(xt2, p['theta_t1'], xi2, p['theta_i1'],
                             dinv, p['bias_t0'], p['bias_i0'])
    et2_raw, ei2_raw = _seg_s1(yt2, yi2, pk_e, cnt_e)
    et2, ei2 = _escale(et2_raw, ei2_raw, binv)
    xt3, xi3 = _seg_s2(et2, ei2, pk_n, cnt_n)
    tg0_raw = xt3[::LENS]
    ig0_raw = xi3[::LENS]

    d0 = dinv[::LENS].reshape(BSZ, 1)
    return _head(tg0_raw, ig0_raw, d0, p['bias_t1'], p['bias_i1'], user, p)


# speedup vs baseline: 8.1350x; 1.0123x over previous
import functools

import jax
import jax.numpy as jnp
import numpy as np
from jax import lax
from jax.experimental import pallas as pl
from jax.experimental.pallas import tpu as pltpu
from jax.experimental.pallas import tpu_sc as plsc

BSZ = 64
LENS = 300
EMB = 128
HALF = EMB // 2
D = EMB + HALF
N_NODES = BSZ * LENS
E_INC = 307200
N_ITEMS = 20000
HEADS = 4
DH = D // HEADS

NC, NS = 2, 16
EPT = E_INC // NS
ICH = 128
NCH = EPT // ICH
RNG = N_NODES // NC
NDUM = 16
RWT = RNG // NS
XCH = 120
BPT = RWT // LENS


def _make_pos():
    position = np.arange(LENS)[:, None].astype(np.float64)
    div_term = np.exp(np.arange(0, EMB, 2) * (-np.log(10000.0) / EMB))
    pe = np.zeros((LENS + 1, EMB))
    pe[1:, 0::2] = np.sin(position * div_term)
    pe[1:, 1::2] = np.cos(position * div_term)
    return np.ascontiguousarray((pe[:LENS] * 0.002).astype(np.float32))


_POS = _make_pos()


def _proj_body(tab_ref, w_ref, b_ref, o_ref):
    o_ref[...] = jnp.dot(tab_ref[...], w_ref[...],
                         preferred_element_type=jnp.float32) + b_ref[...]


def _project_table(table, w, b, rows_blk):
    n, k = table.shape
    return pl.pallas_call(
        _proj_body,
        grid=(n // rows_blk,),
        in_specs=[
            pl.BlockSpec((rows_blk, k), lambda i: (i, 0)),
            pl.BlockSpec((k, EMB), lambda i: (0, 0)),
            pl.BlockSpec((1, EMB), lambda i: (0, 0)),
        ],
        out_specs=pl.BlockSpec((rows_blk, EMB), lambda i: (i, 0)),
        out_shape=jax.ShapeDtypeStruct((n, EMB), jnp.float32),
    )(table, w, b.reshape(1, EMB))


_MMBLK = 1200


def _mm2_body(xt_ref, tht_ref, xi_ref, thi_ref, yt_ref, yi_ref):
    yt_ref[...] = jnp.dot(xt_ref[...], tht_ref[...],
                          preferred_element_type=jnp.float32)
    yi_ref[...] = jnp.dot(xi_ref[...], thi_ref[...],
                          preferred_element_type=jnp.float32)


def _mm2_scaled_body(xt_ref, tht_ref, xi_ref, thi_ref, d_ref,
                     bt_ref, bi_ref, yt_ref, yi_ref):
    d = d_ref[...]
    xt = xt_ref[...] * d + bt_ref[...]
    xi = xi_ref[...] * d + bi_ref[...]
    yt_ref[...] = jnp.dot(xt, tht_ref[...], preferred_element_type=jnp.float32)
    yi_ref[...] = jnp.dot(xi, thi_ref[...], preferred_element_type=jnp.float32)


def _layer_matmul(xt, tht, xi, thi, dinv=None, bt=None, bi=None):
    xspec = pl.BlockSpec((_MMBLK, EMB), lambda i: (i, 0))
    tspec = pl.BlockSpec((EMB, EMB), lambda i: (0, 0))
    bspec = pl.BlockSpec((1, EMB), lambda i: (0, 0))
    dspec = pl.BlockSpec((_MMBLK, 1), lambda i: (i, 0))
    out_shape = [jax.ShapeDtypeStruct((N_NODES, EMB), jnp.float32)] * 2
    if dinv is None:
        return pl.pallas_call(
            _mm2_body,
            grid=(N_NODES // _MMBLK,),
            in_specs=[xspec, tspec, xspec, tspec],
            out_specs=[xspec, xspec],
            out_shape=out_shape,
        )(xt, tht, xi, thi)
    return pl.pallas_call(
        _mm2_scaled_body,
        grid=(N_NODES // _MMBLK,),
        in_specs=[xspec, tspec, xspec, tspec, dspec, bspec, bspec],
        out_specs=[xspec, xspec],
        out_shape=out_shape,
    )(xt, tht, xi, thi, dinv.reshape(N_NODES, 1),
      bt.reshape(1, EMB), bi.reshape(1, EMB))


def _escale_body(et_ref, ei_ref, b_ref, ot_ref, oi_ref):
    b = b_ref[...]
    ot_ref[...] = et_ref[...] * b
    oi_ref[...] = ei_ref[...] * b


def _escale(et, ei, binv):
    xspec = pl.BlockSpec((_MMBLK, EMB), lambda i: (i, 0))
    dspec = pl.BlockSpec((_MMBLK, 1), lambda i: (i, 0))
    return pl.pallas_call(
        _escale_body,
        grid=(N_NODES // _MMBLK,),
        in_specs=[xspec, xspec, dspec],
        out_specs=[xspec, xspec],
        out_shape=[jax.ShapeDtypeStruct((N_NODES, EMB), jnp.float32)] * 2,
    )(et, ei, binv.reshape(N_NODES, 1))


def _head_body(tg0_ref, ig0_ref, d0_ref, bt1_ref, bi1_ref,
               user_ref, w3_ref, b3_ref,
               wq_ref, bq_ref, wk_ref, bk_ref, wv_ref, bv_ref,
               wo_ref, bo_ref, wf1_ref, bf1_ref, wf2_ref, bf2_ref,
               ln1g_ref, ln1b_ref, ln2g_ref, ln2b_ref,
               dw_ref, db_ref, w4_ref, b4_ref, out_ref):
    f32 = jnp.float32

    def mm(a, b):
        return jnp.dot(a, b, preferred_element_type=f32)

    def layer_norm(x, g, b):
        m = jnp.mean(x, axis=-1, keepdims=True)
        v = jnp.mean((x - m) * (x - m), axis=-1, keepdims=True)
        return (x - m) / jnp.sqrt(v + 1e-5) * g + b

    d0 = d0_ref[...]
    tg0 = tg0_ref[...] * d0 + bt1_ref[...]
    ig0 = ig0_ref[...] * d0 + bi1_ref[...]
    user = user_ref[...][:, :HALF]
    text_user = jnp.concatenate([tg0, user], axis=1)
    img_user = jnp.concatenate([ig0, user], axis=1)
    tiu = mm(jnp.concatenate([text_user, img_user], axis=1),
             w3_ref[...]) + b3_ref[...]
    q = mm(tiu, wq_ref[...]) + bq_ref[...]
    k1 = mm(text_user, wk_ref[...]) + bk_ref[...]
    k2 = mm(img_user, wk_ref[...]) + bk_ref[...]
    v1 = mm(text_user, wv_ref[...]) + bv_ref[...]
    v2 = mm(img_user, wv_ref[...]) + bv_ref[...]
    scale = np.float32(1.0 / np.sqrt(DH))
    ao_parts = []
    for h in range(HEADS):
        sl = slice(h * DH, (h + 1) * DH)
        qh, k1h, k2h = q[:, sl], k1[:, sl], k2[:, sl]
        s1 = jnp.sum(qh * k1h, axis=1, keepdims=True) * scale
        s2 = jnp.sum(qh * k2h, axis=1, keepdims=True) * scale
        m = jnp.maximum(s1, s2)
        e1 = jnp.exp(s1 - m)
        e2 = jnp.exp(s2 - m)
        tot = e1 + e2
        ao_parts.append((e1 / tot) * v1[:, sl] + (e2 / tot) * v2[:, sl])
    ao = mm(jnp.concatenate(ao_parts, axis=1), wo_ref[...]) + bo_ref[...]
    x = layer_norm(tiu + ao, ln1g_ref[...], ln1b_ref[...])
    ff = mm(jnp.maximum(mm(x, wf1_ref[...]) + bf1_ref[...], 0.0),
            wf2_ref[...]) + bf2_ref[...]
    x = layer_norm(x + ff, ln2g_ref[...], ln2b_ref[...])
    x = jnp.maximum(mm(x, dw_ref[...]) + db_ref[...], 0.0)
    out_ref[...] = mm(x, w4_ref[...]) + b4_ref[...]


def _head(tg0, ig0, d0, bt1, bi1, user, p):
    w4p = jnp.pad(p['W4'], ((0, 0), (0, 127)))
    b4p = jnp.pad(p['b4'], (0, 127)).reshape(1, 128)
    args = [tg0, ig0, d0, bt1.reshape(1, EMB), bi1.reshape(1, EMB), user,
            p['W3'], p['b3'].reshape(1, D),
            p['Wq'], p['bq'].reshape(1, D), p['Wk'], p['bk'].reshape(1, D),
            p['Wv'], p['bv'].reshape(1, D), p['Wo'], p['bo'].reshape(1, D),
            p['Wf1'], p['bf1'].reshape(1, D), p['Wf2'], p['bf2'].reshape(1, D),
            p['ln1_g'].reshape(1, D), p['ln1_b'].reshape(1, D),
            p['ln2_g'].reshape(1, D), p['ln2_b'].reshape(1, D),
            p['dW'], p['db'].reshape(1, D), w4p, b4p]
    out = pl.pallas_call(
        _head_body,
        out_shape=jax.ShapeDtypeStruct((BSZ, 128), jnp.float32),
    )(*args)
    return out[:, :1]


_MESH = plsc.VectorSubcoreMesh(
    core_axis_name="c", subcore_axis_name="s", num_cores=NC, num_subcores=NS)


_PADK = 0x7FFF


@functools.partial(
    pl.kernel,
    out_type=[
        jax.ShapeDtypeStruct((N_NODES, EMB), jnp.float32),
        jax.ShapeDtypeStruct((N_NODES, EMB), jnp.float32),
        jax.ShapeDtypeStruct((BSZ, EMB), jnp.float32),
        jax.ShapeDtypeStruct((N_NODES,), jnp.float32),
        jax.ShapeDtypeStruct((N_NODES,), jnp.float32),
        jax.ShapeDtypeStruct((NC, NS, EPT), jnp.int32),
        jax.ShapeDtypeStruct((NC, NS, 16), jnp.int32),
        jax.ShapeDtypeStruct((NC, NS, EPT), jnp.int32),
        jax.ShapeDtypeStruct((NC, NS, 16), jnp.int32),
    ],
    mesh=_MESH,
    compiler_params=pltpu.CompilerParams(needs_layout_passes=False),
    scratch_types=[
        pltpu.VMEM((XCH,), jnp.int32),
        pltpu.VMEM((XCH, EMB), jnp.float32),
        pltpu.VMEM((LENS, EMB), jnp.float32),
        pltpu.VMEM_SHARED((N_NODES,), jnp.float32),
        pltpu.VMEM((RWT * 2,), jnp.float32),
        pltpu.VMEM((ICH,), jnp.float32),
        pltpu.VMEM((ICH,), jnp.int32),
        pltpu.VMEM((ICH,), jnp.int32),
        pltpu.VMEM((EPT,), jnp.int32),
        pltpu.VMEM((EPT,), jnp.int32),
        pltpu.VMEM((16,), jnp.int32),
        pltpu.VMEM((8,), jnp.int32),
        pltpu.VMEM((8, EMB), jnp.float32),
        pltpu.SemaphoreType.DMA,
    ],
)
def _sc_prep(pt_hbm, pi_hbm, ri_hbm, node_hbm, edge_hbm, uid_hbm, ut_hbm,
             pos_hbm, xt_hbm, xi_hbm, user_hbm, dinv_hbm, binv_hbm,
             pke_hbm, cnte_hbm, pkn_hbm, cntn_hbm,
             idxv, rows, posv, deg, degv, onesv, nbv, ebv, pebuf, pnbuf,
             cntv, uidv, urows, sem):
    c = lax.axis_index("c")
    s = lax.axis_index("s")
    w = c * NS + s
    nb = w * (N_NODES // (NC * NS))
    dpt = RWT * 2
    lo = c * RNG
    hi = lo + RNG

    def zdeg(j, _):
        degv[pl.ds(j * 16, 16)] = jnp.zeros((16,), jnp.float32)
        return 0
    lax.fori_loop(0, dpt // 16, zdeg, 0)
    pltpu.sync_copy(degv, deg.at[pl.ds(s * dpt, dpt)])
    for j in range(ICH // 16):
        onesv[pl.ds(j * 16, 16)] = jnp.ones((16,), jnp.float32)

    lane = lax.iota(jnp.int32, 16)
    pad_e = lane | (_PADK << 15)
    pad_n = _PADK | (lane << 15)

    def zpk(j, _):
        sl = pl.ds(j * 16, 16)
        pebuf[sl] = pad_e
        pnbuf[sl] = pad_n
        return 0
    lax.fori_loop(0, EPT // 16, zpk, 0)
    plsc.subcore_barrier()

    def part_body(ch, carry):
        ce, cn = carry
        base = s * EPT + ch * ICH
        pltpu.sync_copy(node_hbm.at[pl.ds(base, ICH)], nbv)
        pltpu.sync_copy(edge_hbm.at[pl.ds(base, ICH)], ebv)

        @pl.when(c == 0)
        def _():
            pltpu.sync_copy(onesv, deg.at[nbv], add=True)

        @pl.when(c == 1)
        def _():
            pltpu.sync_copy(onesv, deg.at[ebv], add=True)

        one = jnp.ones((16,), jnp.int32)
        zero = jnp.zeros((16,), jnp.int32)
        for j in range(ICH // 16):
            sl = pl.ds(j * 16, 16)
            nv = nbv[sl]
            ev = ebv[sl]
            pk = nv | (ev << 15)
            me = (ev >= lo) & (ev < hi)
            cse = plsc.cumsum(jnp.where(me, one, zero))
            plsc.store_scatter(pebuf, [ce + cse - 1], pk, mask=me)
            ce = ce + jnp.max(cse)
            mn = (nv >= lo) & (nv < hi)
            csn = plsc.cumsum(jnp.where(mn, one, zero))
            plsc.store_scatter(pnbuf, [cn + csn - 1], pk, mask=mn)
            cn = cn + jnp.max(csn)
        return ce, cn

    ce, cn = lax.fori_loop(0, NCH, part_body, (jnp.int32(0), jnp.int32(0)))

    qe = jnp.maximum(jnp.minimum((ce + 383) // 384 * 384, EPT) // 384, 1)
    qn = jnp.maximum(jnp.minimum((cn + 383) // 384 * 384, EPT) // 384, 1)
    cntv[pl.ds(0, 16)] = jnp.full((16,), qe, jnp.int32)
    pltpu.sync_copy(cntv, cnte_hbm.at[c, s])
    pltpu.sync_copy(pebuf, pke_hbm.at[c, s])
    cntv[pl.ds(0, 16)] = jnp.full((16,), qn, jnp.int32)
    pltpu.sync_copy(cntv, cntn_hbm.at[c, s])
    pltpu.sync_copy(pnbuf, pkn_hbm.at[c, s])

    pltpu.sync_copy(pos_hbm, posv)
    for p_hbm, x_hbm in ((pt_hbm, xt_hbm), (pi_hbm, xi_hbm)):
        def xbody(k, _):
            pltpu.sync_copy(ri_hbm.at[pl.ds(nb + k * XCH, XCH)], idxv)
            pltpu.async_copy(p_hbm.at[idxv], rows, sem).wait()

            def posadd(r, _):
                pr = lax.rem(k * XCH + r, LENS)
                for j in range(EMB // 16):
                    sl = pl.ds(j * 16, 16)
                    rows[r, sl] = rows[r, sl] + posv[pr, sl]
                return 0
            lax.fori_loop(0, XCH, posadd, 0)
            pltpu.sync_copy(rows, x_hbm.at[pl.ds(nb + k * XCH, XCH)])
            return 0
        lax.fori_loop(0, (N_NODES // (NC * NS)) // XCH, xbody, 0)

    @pl.when((c == 0) & (s < 8))
    def _():
        pltpu.sync_copy(uid_hbm.at[pl.ds(s * 8, 8)], uidv)
        pltpu.async_copy(ut_hbm.at[uidv], urows, sem).wait()
        pltpu.sync_copy(urows, user_hbm.at[pl.ds(s * 8, 8)])

    plsc.subcore_barrier()

    pltpu.sync_copy(deg.at[pl.ds(s * dpt, dpt)], degv)

    def inv(j, _):
        sl = pl.ds(j * 16, 16)
        v = degv[sl]
        degv[sl] = jnp.where(v > 0.0, 1.0 / v, 0.0)
        return 0
    lax.fori_loop(0, dpt // 16, inv, 0)

    @pl.when(c == 0)
    def _():
        pltpu.sync_copy(degv, dinv_hbm.at[pl.ds(s * dpt, dpt)])

    @pl.when(c == 1)
    def _():
        pltpu.sync_copy(degv, binv_hbm.at[pl.ds(s * dpt, dpt)])


def _make_seg(key_low):
    @functools.partial(
        pl.kernel,
        out_type=[
            jax.ShapeDtypeStruct((N_NODES, EMB), jnp.float32),
            jax.ShapeDtypeStruct((N_NODES, EMB), jnp.float32),
        ],
        mesh=_MESH,
        compiler_params=pltpu.CompilerParams(needs_layout_passes=False),
        scratch_types=[
            pltpu.VMEM_SHARED((RNG + NDUM, EMB), jnp.float32),
            pltpu.VMEM((ICH,), jnp.int32),
            pltpu.VMEM((ICH,), jnp.int32),
            pltpu.VMEM((ICH,), jnp.int32),
            pltpu.VMEM((ICH,), jnp.int32),
            pltpu.VMEM((ICH,), jnp.int32),
            pltpu.VMEM((ICH,), jnp.int32),
            pltpu.VMEM((ICH, EMB), jnp.float32),
            pltpu.VMEM((ICH, EMB), jnp.float32),
            pltpu.VMEM((ICH, EMB), jnp.float32),
            pltpu.VMEM((24, EMB), jnp.float32),
            pltpu.VMEM((16,), jnp.int32),
            pltpu.SemaphoreType.DMA,
            pltpu.SemaphoreType.DMA,
            pltpu.SemaphoreType.DMA,
            pltpu.SemaphoreType.DMA,
            pltpu.SemaphoreType.DMA,
        ],
    )
    def seg(gt_hbm, gi_hbm, pk_hbm, cnt_hbm, ot_hbm, oi_hbm,
            acc, nvA, nvB, nvC, evcA, evcB, evcC, rowsA, rowsB, rowsC,
            zbuf, cbuf, gsA, gsB, gsC, ss, zs):
        c = lax.axis_index("c")
        s = lax.axis_index("s")
        lo = c * RNG
        r0 = s * RWT
        bufs = ((nvA, evcA, rowsA, gsA),
                (nvB, evcB, rowsB, gsB),
                (nvC, evcC, rowsC, gsC))
        nbuf = len(bufs)

        def zb(r, _):
            for j in range(EMB // 16):
                zbuf[r, pl.ds(j * 16, 16)] = jnp.zeros((16,), jnp.float32)
            return 0
        lax.fori_loop(0, 24, zb, 0)

        pltpu.sync_copy(cnt_hbm.at[c, s], cbuf)
        qtrips = jnp.max(cbuf[pl.ds(0, 16)])

        def load_idx_and_gather(cx, nv, evc, rows, gs, g_hbm):
            pltpu.sync_copy(pk_hbm.at[c, s, pl.ds(cx * ICH, ICH)], nv)
            for j in range(ICH // 16):
                sl = pl.ds(j * 16, 16)
                v = nv[sl]
                if key_low:
                    g, e = v >> 15, v & _PADK
                else:
                    g, e = v & _PADK, v >> 15
                loc = e - lo
                dummy = RNG + (g & (NDUM - 1))
                ok = (loc >= 0) & (loc < RNG)
                evc[sl] = jnp.where(ok, loc, dummy)
                nv[sl] = g
            pltpu.async_copy(g_hbm.at[nv], rows, gs)

        for m, (g_hbm, o_hbm) in enumerate(((gt_hbm, ot_hbm),
                                            (gi_hbm, oi_hbm))):
            for k in range(RWT // 24):
                pltpu.async_copy(zbuf, acc.at[pl.ds(r0 + k * 24, 24)], zs)
            for k in range(RWT // 24):
                pltpu.make_async_copy(
                    zbuf, acc.at[pl.ds(r0 + k * 24, 24)], zs).wait()
            plsc.subcore_barrier()

            load_idx_and_gather(0, *bufs[0], g_hbm)
            load_idx_and_gather(1, *bufs[1], g_hbm)

            def body(q, _):
                for x in range(nbuf):
                    cx = nbuf * q + x
                    nv, evc, rows, gs = bufs[x]
                    nv2, evc2, rows2, gs2 = bufs[(x + 2) % nbuf]
                    if x == 0:
                        @pl.when(q >= 1)
                        def _():
                            pltpu.make_async_copy(
                                rows2, acc.at[evc2], ss).wait()
                    else:
                        pltpu.make_async_copy(
                            bufs[x - 1][2], acc.at[bufs[x - 1][1]],
                            ss).wait()

                    @pl.when(cx + 2 < nbuf * qtrips)
                    def _():
                        load_idx_and_gather(cx + 2, nv2, evc2, rows2, gs2,
                                            g_hbm)
                    pltpu.make_async_copy(g_hbm.at[nv], rows, gs).wait()
                    pltpu.async_copy(rows, acc.at[evc], ss, add=True)
                return 0
            lax.fori_loop(0, qtrips, body, 0)
            pltpu.make_async_copy(rowsC, acc.at[evcC], ss).wait()
            plsc.subcore_barrier()

            pltpu.sync_copy(acc.at[pl.ds(r0, RWT)],
                            o_hbm.at[pl.ds(lo + r0, RWT)])
            if m == 0:
                plsc.subcore_barrier()

    return seg


_seg_s1 = _make_seg(False)
_seg_s2 = _make_seg(True)


def kernel(input, hg_idx, related_items, label, uid, params):
    p = params
    node = hg_idx[0]
    edge = hg_idx[1]

    pt = _project_table(p['text_table'], p['W1'], p['b1'], 400)
    pi = _project_table(p['img_table'], p['W2'], p['b2'], 400)

    pos = jnp.asarray(_POS)
    ut_p = jnp.pad(p['user_table'], ((0, 0), (0, EMB - HALF)))
    (xt, xi, user, dinv, binv,
     pk_e, cnt_e, pk_n, cnt_n) = _sc_prep(
        pt, pi, related_items, node, edge, uid, ut_p, pos)

    yt, yi = _layer_matmul(xt, p['theta_t0'], xi, p['theta_i0'])
    et_raw, ei_raw = _seg_s1(yt, yi, pk_e, cnt_e)
    et, ei = _escale(et_raw, ei_raw, binv)
    xt2, xi2 = _seg_s2(et, ei, pk_n, cnt_n)

    yt2, yi2 = _layer_matmul(xt2, p['theta_t1'], xi2, p['theta_i1'],
                             dinv, p['bias_t0'], p['bias_i0'])
    et2_raw, ei2_raw = _seg_s1(yt2, yi2, pk_e, cnt_e)
    et2, ei2 = _escale(et2_raw, ei2_raw, binv)
    xt3, xi3 = _seg_s2(et2, ei2, pk_n, cnt_n)
    tg0_raw = xt3[::LENS]
    ig0_raw = xi3[::LENS]

    d0 = dinv[::LENS].reshape(BSZ, 1)
    return _head(tg0_raw, ig0_raw, d0, p['bias_t1'], p['bias_i1'], user, p)


# speedup vs baseline: 9.1840x; 1.1289x over previous
import functools

import jax
import jax.numpy as jnp
import numpy as np
from jax import lax
from jax.experimental import pallas as pl
from jax.experimental.pallas import tpu as pltpu
from jax.experimental.pallas import tpu_sc as plsc

BSZ = 64
LENS = 300
EMB = 128
HALF = EMB // 2
D = EMB + HALF
N_NODES = BSZ * LENS
E_INC = 307200
N_ITEMS = 20000
HEADS = 4
DH = D // HEADS

NC, NS = 2, 16
EPT = E_INC // NS
ICH = 128
NCH = EPT // ICH
RNG = N_NODES // NC
NDUM = 16
RWT = RNG // NS
XCH = 120
BPT = RWT // LENS


def _make_pos():
    position = np.arange(LENS)[:, None].astype(np.float64)
    div_term = np.exp(np.arange(0, EMB, 2) * (-np.log(10000.0) / EMB))
    pe = np.zeros((LENS + 1, EMB))
    pe[1:, 0::2] = np.sin(position * div_term)
    pe[1:, 1::2] = np.cos(position * div_term)
    return np.ascontiguousarray((pe[:LENS] * 0.002).astype(np.float32))


_POS = _make_pos()


def _proj_body(tab_ref, w_ref, b_ref, o_ref):
    o_ref[...] = jnp.dot(tab_ref[...], w_ref[...],
                         preferred_element_type=jnp.float32) + b_ref[...]


def _project_table(table, w, b, rows_blk):
    n, k = table.shape
    return pl.pallas_call(
        _proj_body,
        grid=(n // rows_blk,),
        in_specs=[
            pl.BlockSpec((rows_blk, k), lambda i: (i, 0)),
            pl.BlockSpec((k, EMB), lambda i: (0, 0)),
            pl.BlockSpec((1, EMB), lambda i: (0, 0)),
        ],
        out_specs=pl.BlockSpec((rows_blk, EMB), lambda i: (i, 0)),
        out_shape=jax.ShapeDtypeStruct((n, EMB), jnp.float32),
    )(table, w, b.reshape(1, EMB))


_MMBLK = 1200


def _mm2_pos_body(xt_ref, tht_ref, xi_ref, thi_ref, pos_ref, yt_ref, yi_ref):
    pos4 = jnp.concatenate([pos_ref[...]] * (_MMBLK // LENS), axis=0)
    yt_ref[...] = jnp.dot(xt_ref[...] + pos4, tht_ref[...],
                          preferred_element_type=jnp.float32)
    yi_ref[...] = jnp.dot(xi_ref[...] + pos4, thi_ref[...],
                          preferred_element_type=jnp.float32)


def _mm2_scaled_body(xt_ref, tht_ref, xi_ref, thi_ref, d_ref,
                     bt_ref, bi_ref, yt_ref, yi_ref):
    d = d_ref[...]
    xt = xt_ref[...] * d + bt_ref[...]
    xi = xi_ref[...] * d + bi_ref[...]
    yt_ref[...] = jnp.dot(xt, tht_ref[...], preferred_element_type=jnp.float32)
    yi_ref[...] = jnp.dot(xi, thi_ref[...], preferred_element_type=jnp.float32)


def _layer_matmul(xt, tht, xi, thi, pos=None, dinv=None, bt=None, bi=None):
    xspec = pl.BlockSpec((_MMBLK, EMB), lambda i: (i, 0))
    tspec = pl.BlockSpec((EMB, EMB), lambda i: (0, 0))
    bspec = pl.BlockSpec((1, EMB), lambda i: (0, 0))
    dspec = pl.BlockSpec((_MMBLK, 1), lambda i: (i, 0))
    pspec = pl.BlockSpec((LENS, EMB), lambda i: (0, 0))
    out_shape = [jax.ShapeDtypeStruct((N_NODES, EMB), jnp.float32)] * 2
    if dinv is None:
        return pl.pallas_call(
            _mm2_pos_body,
            grid=(N_NODES // _MMBLK,),
            in_specs=[xspec, tspec, xspec, tspec, pspec],
            out_specs=[xspec, xspec],
            out_shape=out_shape,
        )(xt, tht, xi, thi, pos)
    return pl.pallas_call(
        _mm2_scaled_body,
        grid=(N_NODES // _MMBLK,),
        in_specs=[xspec, tspec, xspec, tspec, dspec, bspec, bspec],
        out_specs=[xspec, xspec],
        out_shape=out_shape,
    )(xt, tht, xi, thi, dinv.reshape(N_NODES, 1),
      bt.reshape(1, EMB), bi.reshape(1, EMB))


def _escale_body(et_ref, ei_ref, b_ref, ot_ref, oi_ref):
    b = b_ref[...]
    ot_ref[...] = et_ref[...] * b
    oi_ref[...] = ei_ref[...] * b


def _escale(et, ei, binv):
    xspec = pl.BlockSpec((_MMBLK, EMB), lambda i: (i, 0))
    dspec = pl.BlockSpec((_MMBLK, 1), lambda i: (i, 0))
    return pl.pallas_call(
        _escale_body,
        grid=(N_NODES // _MMBLK,),
        in_specs=[xspec, xspec, dspec],
        out_specs=[xspec, xspec],
        out_shape=[jax.ShapeDtypeStruct((N_NODES, EMB), jnp.float32)] * 2,
    )(et, ei, binv.reshape(N_NODES, 1))


def _head_body(tg0_ref, ig0_ref, d0_ref, bt1_ref, bi1_ref,
               user_ref, w3_ref, b3_ref,
               wq_ref, bq_ref, wk_ref, bk_ref, wv_ref, bv_ref,
               wo_ref, bo_ref, wf1_ref, bf1_ref, wf2_ref, bf2_ref,
               ln1g_ref, ln1b_ref, ln2g_ref, ln2b_ref,
               dw_ref, db_ref, w4_ref, b4_ref, out_ref):
    f32 = jnp.float32

    def mm(a, b):
        return jnp.dot(a, b, preferred_element_type=f32)

    def layer_norm(x, g, b):
        m = jnp.mean(x, axis=-1, keepdims=True)
        v = jnp.mean((x - m) * (x - m), axis=-1, keepdims=True)
        return (x - m) / jnp.sqrt(v + 1e-5) * g + b

    d0 = d0_ref[...]
    tg0 = tg0_ref[...] * d0 + bt1_ref[...]
    ig0 = ig0_ref[...] * d0 + bi1_ref[...]
    user = user_ref[...][:, :HALF]
    text_user = jnp.concatenate([tg0, user], axis=1)
    img_user = jnp.concatenate([ig0, user], axis=1)
    tiu = mm(jnp.concatenate([text_user, img_user], axis=1),
             w3_ref[...]) + b3_ref[...]
    q = mm(tiu, wq_ref[...]) + bq_ref[...]
    k1 = mm(text_user, wk_ref[...]) + bk_ref[...]
    k2 = mm(img_user, wk_ref[...]) + bk_ref[...]
    v1 = mm(text_user, wv_ref[...]) + bv_ref[...]
    v2 = mm(img_user, wv_ref[...]) + bv_ref[...]
    scale = np.float32(1.0 / np.sqrt(DH))
    ao_parts = []
    for h in range(HEADS):
        sl = slice(h * DH, (h + 1) * DH)
        qh, k1h, k2h = q[:, sl], k1[:, sl], k2[:, sl]
        s1 = jnp.sum(qh * k1h, axis=1, keepdims=True) * scale
        s2 = jnp.sum(qh * k2h, axis=1, keepdims=True) * scale
        m = jnp.maximum(s1, s2)
        e1 = jnp.exp(s1 - m)
        e2 = jnp.exp(s2 - m)
        tot = e1 + e2
        ao_parts.append((e1 / tot) * v1[:, sl] + (e2 / tot) * v2[:, sl])
    ao = mm(jnp.concatenate(ao_parts, axis=1), wo_ref[...]) + bo_ref[...]
    x = layer_norm(tiu + ao, ln1g_ref[...], ln1b_ref[...])
    ff = mm(jnp.maximum(mm(x, wf1_ref[...]) + bf1_ref[...], 0.0),
            wf2_ref[...]) + bf2_ref[...]
    x = layer_norm(x + ff, ln2g_ref[...], ln2b_ref[...])
    x = jnp.maximum(mm(x, dw_ref[...]) + db_ref[...], 0.0)
    out_ref[...] = mm(x, w4_ref[...]) + b4_ref[...]


def _head(tg0, ig0, d0, bt1, bi1, user, p):
    w4p = jnp.pad(p['W4'], ((0, 0), (0, 127)))
    b4p = jnp.pad(p['b4'], (0, 127)).reshape(1, 128)
    args = [tg0, ig0, d0, bt1.reshape(1, EMB), bi1.reshape(1, EMB), user,
            p['W3'], p['b3'].reshape(1, D),
            p['Wq'], p['bq'].reshape(1, D), p['Wk'], p['bk'].reshape(1, D),
            p['Wv'], p['bv'].reshape(1, D), p['Wo'], p['bo'].reshape(1, D),
            p['Wf1'], p['bf1'].reshape(1, D), p['Wf2'], p['bf2'].reshape(1, D),
            p['ln1_g'].reshape(1, D), p['ln1_b'].reshape(1, D),
            p['ln2_g'].reshape(1, D), p['ln2_b'].reshape(1, D),
            p['dW'], p['db'].reshape(1, D), w4p, b4p]
    out = pl.pallas_call(
        _head_body,
        out_shape=jax.ShapeDtypeStruct((BSZ, 128), jnp.float32),
    )(*args)
    return out[:, :1]


_MESH = plsc.VectorSubcoreMesh(
    core_axis_name="c", subcore_axis_name="s", num_cores=NC, num_subcores=NS)


_PADK = 0x7FFF


@functools.partial(
    pl.kernel,
    out_type=[
        jax.ShapeDtypeStruct((N_NODES, EMB), jnp.float32),
        jax.ShapeDtypeStruct((N_NODES, EMB), jnp.float32),
        jax.ShapeDtypeStruct((BSZ, EMB), jnp.float32),
        jax.ShapeDtypeStruct((N_NODES,), jnp.float32),
        jax.ShapeDtypeStruct((N_NODES,), jnp.float32),
        jax.ShapeDtypeStruct((NC, NS, EPT), jnp.int32),
        jax.ShapeDtypeStruct((NC, NS, 16), jnp.int32),
        jax.ShapeDtypeStruct((NC, NS, EPT), jnp.int32),
        jax.ShapeDtypeStruct((NC, NS, 16), jnp.int32),
    ],
    mesh=_MESH,
    compiler_params=pltpu.CompilerParams(needs_layout_passes=False),
    scratch_types=[
        pltpu.VMEM((XCH,), jnp.int32),
        pltpu.VMEM((XCH, EMB), jnp.float32),
        pltpu.VMEM_SHARED((N_NODES,), jnp.float32),
        pltpu.VMEM((RWT * 2,), jnp.float32),
        pltpu.VMEM((ICH,), jnp.float32),
        pltpu.VMEM((ICH,), jnp.int32),
        pltpu.VMEM((ICH,), jnp.int32),
        pltpu.VMEM((ICH,), jnp.int32),
        pltpu.VMEM((ICH,), jnp.int32),
        pltpu.VMEM((EPT,), jnp.int32),
        pltpu.VMEM((EPT,), jnp.int32),
        pltpu.VMEM((16,), jnp.int32),
        pltpu.VMEM((8,), jnp.int32),
        pltpu.VMEM((8, EMB), jnp.float32),
        pltpu.SemaphoreType.DMA,
        pltpu.SemaphoreType.DMA,
        pltpu.SemaphoreType.DMA,
        pltpu.SemaphoreType.DMA,
        pltpu.SemaphoreType.DMA,
    ],
)
def _sc_prep(pt_hbm, pi_hbm, ri_hbm, node_hbm, edge_hbm, uid_hbm, ut_hbm,
             xt_hbm, xi_hbm, user_hbm, dinv_hbm, binv_hbm,
             pke_hbm, cnte_hbm, pkn_hbm, cntn_hbm,
             idxv, rows, deg, degv, onesv, nbvA, ebvA, nbvB, ebvB,
             pebuf, pnbuf, cntv, uidv, urows, sem, isA, isB, dsA, dsB):
    c = lax.axis_index("c")
    s = lax.axis_index("s")
    w = c * NS + s
    nb = w * (N_NODES // (NC * NS))
    dpt = RWT * 2
    lo = c * RNG
    hi = lo + RNG

    def zdeg(j, _):
        degv[pl.ds(j * 16, 16)] = jnp.zeros((16,), jnp.float32)
        return 0
    lax.fori_loop(0, dpt // 16, zdeg, 0)
    pltpu.sync_copy(degv, deg.at[pl.ds(s * dpt, dpt)])
    for j in range(ICH // 16):
        onesv[pl.ds(j * 16, 16)] = jnp.ones((16,), jnp.float32)

    lane = lax.iota(jnp.int32, 16)
    pad_e = lane | (_PADK << 15)
    pad_n = _PADK | (lane << 15)

    def zpk(j, _):
        sl = pl.ds(j * 16, 16)
        pebuf[sl] = pad_e
        pnbuf[sl] = pad_n
        return 0
    lax.fori_loop(0, EPT // 16, zpk, 0)
    plsc.subcore_barrier()

    pbufs = ((nbvA, ebvA, isA, dsA), (nbvB, ebvB, isB, dsB))
    pltpu.sync_copy(node_hbm.at[pl.ds(s * EPT, ICH)], nbvA)
    pltpu.sync_copy(edge_hbm.at[pl.ds(s * EPT, ICH)], ebvA)
    pltpu.async_copy(node_hbm.at[pl.ds(s * EPT + ICH, ICH)], nbvB, isB)
    pltpu.async_copy(edge_hbm.at[pl.ds(s * EPT + ICH, ICH)], ebvB, isB)

    def part_body(q, carry):
        ce, cn = carry
        for x, (nbv, ebv, isem, dsem) in enumerate(pbufs):
            cx = 2 * q + x

            def wait_idx():
                pltpu.make_async_copy(
                    node_hbm.at[pl.ds(0, ICH)], nbv, isem).wait()
                pltpu.make_async_copy(
                    edge_hbm.at[pl.ds(0, ICH)], ebv, isem).wait()

            if x == 0:
                @pl.when(q >= 1)
                def _():
                    wait_idx()
            else:
                wait_idx()

            @pl.when(c == 0)
            def _():
                pltpu.async_copy(onesv, deg.at[nbv], dsem, add=True)

            @pl.when(c == 1)
            def _():
                pltpu.async_copy(onesv, deg.at[ebv], dsem, add=True)

            one = jnp.ones((16,), jnp.int32)
            zero = jnp.zeros((16,), jnp.int32)
            for j in range(ICH // 16):
                sl = pl.ds(j * 16, 16)
                nv = nbv[sl]
                ev = ebv[sl]
                pk = nv | (ev << 15)
                me = (ev >= lo) & (ev < hi)
                cse = plsc.cumsum(jnp.where(me, one, zero))
                plsc.store_scatter(pebuf, [ce + cse - 1], pk, mask=me)
                ce = ce + jnp.max(cse)
                mn = (nv >= lo) & (nv < hi)
                csn = plsc.cumsum(jnp.where(mn, one, zero))
                plsc.store_scatter(pnbuf, [cn + csn - 1], pk, mask=mn)
                cn = cn + jnp.max(csn)

            @pl.when(c == 0)
            def _():
                pltpu.make_async_copy(onesv, deg.at[nbv], dsem).wait()

            @pl.when(c == 1)
            def _():
                pltpu.make_async_copy(onesv, deg.at[ebv], dsem).wait()

            @pl.when(cx + 2 < NCH)
            def _():
                base = s * EPT + (cx + 2) * ICH
                pltpu.async_copy(node_hbm.at[pl.ds(base, ICH)], nbv, isem)
                pltpu.async_copy(edge_hbm.at[pl.ds(base, ICH)], ebv, isem)
        return ce, cn

    ce, cn = lax.fori_loop(0, NCH // 2, part_body,
                           (jnp.int32(0), jnp.int32(0)))

    qe = jnp.maximum(jnp.minimum((ce + 383) // 384 * 384, EPT) // 384, 1)
    qn = jnp.maximum(jnp.minimum((cn + 383) // 384 * 384, EPT) // 384, 1)
    cntv[pl.ds(0, 16)] = jnp.full((16,), qe, jnp.int32)
    pltpu.sync_copy(cntv, cnte_hbm.at[c, s])
    pltpu.sync_copy(pebuf, pke_hbm.at[c, s])
    cntv[pl.ds(0, 16)] = jnp.full((16,), qn, jnp.int32)
    pltpu.sync_copy(cntv, cntn_hbm.at[c, s])
    pltpu.sync_copy(pnbuf, pkn_hbm.at[c, s])

    for p_hbm, x_hbm in ((pt_hbm, xt_hbm), (pi_hbm, xi_hbm)):
        def xbody(k, _):
            pltpu.sync_copy(ri_hbm.at[pl.ds(nb + k * XCH, XCH)], idxv)
            pltpu.async_copy(p_hbm.at[idxv], rows, sem).wait()
            pltpu.sync_copy(rows, x_hbm.at[pl.ds(nb + k * XCH, XCH)])
            return 0
        lax.fori_loop(0, (N_NODES // (NC * NS)) // XCH, xbody, 0)

    @pl.when((c == 0) & (s < 8))
    def _():
        pltpu.sync_copy(uid_hbm.at[pl.ds(s * 8, 8)], uidv)
        pltpu.async_copy(ut_hbm.at[uidv], urows, sem).wait()
        pltpu.sync_copy(urows, user_hbm.at[pl.ds(s * 8, 8)])

    plsc.subcore_barrier()

    pltpu.sync_copy(deg.at[pl.ds(s * dpt, dpt)], degv)

    def inv(j, _):
        sl = pl.ds(j * 16, 16)
        v = degv[sl]
        degv[sl] = jnp.where(v > 0.0, 1.0 / v, 0.0)
        return 0
    lax.fori_loop(0, dpt // 16, inv, 0)

    @pl.when(c == 0)
    def _():
        pltpu.sync_copy(degv, dinv_hbm.at[pl.ds(s * dpt, dpt)])

    @pl.when(c == 1)
    def _():
        pltpu.sync_copy(degv, binv_hbm.at[pl.ds(s * dpt, dpt)])


def _make_seg(key_low):
    @functools.partial(
        pl.kernel,
        out_type=[
            jax.ShapeDtypeStruct((N_NODES, EMB), jnp.float32),
            jax.ShapeDtypeStruct((N_NODES, EMB), jnp.float32),
        ],
        mesh=_MESH,
        compiler_params=pltpu.CompilerParams(needs_layout_passes=False),
        scratch_types=[
            pltpu.VMEM_SHARED((RNG + NDUM, EMB), jnp.float32),
            pltpu.VMEM((ICH,), jnp.int32),
            pltpu.VMEM((ICH,), jnp.int32),
            pltpu.VMEM((ICH,), jnp.int32),
            pltpu.VMEM((ICH,), jnp.int32),
            pltpu.VMEM((ICH,), jnp.int32),
            pltpu.VMEM((ICH,), jnp.int32),
            pltpu.VMEM((ICH, EMB), jnp.float32),
            pltpu.VMEM((ICH, EMB), jnp.float32),
            pltpu.VMEM((ICH, EMB), jnp.float32),
            pltpu.VMEM((24, EMB), jnp.float32),
            pltpu.VMEM((16,), jnp.int32),
            pltpu.SemaphoreType.DMA,
            pltpu.SemaphoreType.DMA,
            pltpu.SemaphoreType.DMA,
            pltpu.SemaphoreType.DMA,
            pltpu.SemaphoreType.DMA,
        ],
    )
    def seg(gt_hbm, gi_hbm, pk_hbm, cnt_hbm, ot_hbm, oi_hbm,
            acc, nvA, nvB, nvC, evcA, evcB, evcC, rowsA, rowsB, rowsC,
            zbuf, cbuf, gsA, gsB, gsC, ss, zs):
        c = lax.axis_index("c")
        s = lax.axis_index("s")
        lo = c * RNG
        r0 = s * RWT
        bufs = ((nvA, evcA, rowsA, gsA),
                (nvB, evcB, rowsB, gsB),
                (nvC, evcC, rowsC, gsC))
        nbuf = len(bufs)

        def zb(r, _):
            for j in range(EMB // 16):
                zbuf[r, pl.ds(j * 16, 16)] = jnp.zeros((16,), jnp.float32)
            return 0
        lax.fori_loop(0, 24, zb, 0)

        pltpu.sync_copy(cnt_hbm.at[c, s], cbuf)
        qtrips = jnp.max(cbuf[pl.ds(0, 16)])

        def load_idx_and_gather(cx, nv, evc, rows, gs, g_hbm):
            pltpu.sync_copy(pk_hbm.at[c, s, pl.ds(cx * ICH, ICH)], nv)
            for j in range(ICH // 16):
                sl = pl.ds(j * 16, 16)
                v = nv[sl]
                if key_low:
                    g, e = v >> 15, v & _PADK
                else:
                    g, e = v & _PADK, v >> 15
                loc = e - lo
                dummy = RNG + (g & (NDUM - 1))
                ok = (loc >= 0) & (loc < RNG)
                evc[sl] = jnp.where(ok, loc, dummy)
                nv[sl] = g
            pltpu.async_copy(g_hbm.at[nv], rows, gs)

        for m, (g_hbm, o_hbm) in enumerate(((gt_hbm, ot_hbm),
                                            (gi_hbm, oi_hbm))):
            for k in range(RWT // 24):
                pltpu.async_copy(zbuf, acc.at[pl.ds(r0 + k * 24, 24)], zs)
            for k in range(RWT // 24):
                pltpu.make_async_copy(
                    zbuf, acc.at[pl.ds(r0 + k * 24, 24)], zs).wait()
            plsc.subcore_barrier()

            load_idx_and_gather(0, *bufs[0], g_hbm)
            load_idx_and_gather(1, *bufs[1], g_hbm)

            def body(q, _):
                for x in range(nbuf):
                    cx = nbuf * q + x
                    nv, evc, rows, gs = bufs[x]
                    nv2, evc2, rows2, gs2 = bufs[(x + 2) % nbuf]
                    if x == 0:
                        @pl.when(q >= 1)
                        def _():
                            pltpu.make_async_copy(
                                rows2, acc.at[evc2], ss).wait()
                    else:
                        pltpu.make_async_copy(
                            bufs[x - 1][2], acc.at[bufs[x - 1][1]],
                            ss).wait()

                    @pl.when(cx + 2 < nbuf * qtrips)
                    def _():
                        load_idx_and_gather(cx + 2, nv2, evc2, rows2, gs2,
                                            g_hbm)
                    pltpu.make_async_copy(g_hbm.at[nv], rows, gs).wait()
                    pltpu.async_copy(rows, acc.at[evc], ss, add=True)
                return 0
            lax.fori_loop(0, qtrips, body, 0)
            pltpu.make_async_copy(rowsC, acc.at[evcC], ss).wait()
            plsc.subcore_barrier()

            pltpu.sync_copy(acc.at[pl.ds(r0, RWT)],
                            o_hbm.at[pl.ds(lo + r0, RWT)])
            if m == 0:
                plsc.subcore_barrier()

    return seg


_seg_s1 = _make_seg(False)
_seg_s2 = _make_seg(True)


def kernel(input, hg_idx, related_items, label, uid, params):
    p = params
    node = hg_idx[0]
    edge = hg_idx[1]

    pt = _project_table(p['text_table'], p['W1'], p['b1'], 400)
    pi = _project_table(p['img_table'], p['W2'], p['b2'], 400)

    pos = jnp.asarray(_POS)
    ut_p = jnp.pad(p['user_table'], ((0, 0), (0, EMB - HALF)))
    (xt, xi, user, dinv, binv,
     pk_e, cnt_e, pk_n, cnt_n) = _sc_prep(
        pt, pi, related_items, node, edge, uid, ut_p)

    yt, yi = _layer_matmul(xt, p['theta_t0'], xi, p['theta_i0'], pos=pos)
    et_raw, ei_raw = _seg_s1(yt, yi, pk_e, cnt_e)
    et, ei = _escale(et_raw, ei_raw, binv)
    xt2, xi2 = _seg_s2(et, ei, pk_n, cnt_n)

    yt2, yi2 = _layer_matmul(xt2, p['theta_t1'], xi2, p['theta_i1'],
                             dinv=dinv, bt=p['bias_t0'], bi=p['bias_i0'])
    et2_raw, ei2_raw = _seg_s1(yt2, yi2, pk_e, cnt_e)
    et2, ei2 = _escale(et2_raw, ei2_raw, binv)
    xt3, xi3 = _seg_s2(et2, ei2, pk_n, cnt_n)
    tg0_raw = xt3[::LENS]
    ig0_raw = xi3[::LENS]

    d0 = dinv[::LENS].reshape(BSZ, 1)
    return _head(tg0_raw, ig0_raw, d0, p['bias_t1'], p['bias_i1'], user, p)


# speedup vs baseline: 9.3789x; 1.0212x over previous
import functools

import jax
import jax.numpy as jnp
import numpy as np
from jax import lax
from jax.experimental import pallas as pl
from jax.experimental.pallas import tpu as pltpu
from jax.experimental.pallas import tpu_sc as plsc

BSZ = 64
LENS = 300
EMB = 128
HALF = EMB // 2
D = EMB + HALF
N_NODES = BSZ * LENS
E_INC = 307200
N_ITEMS = 20000
TEXT_DIM = 384
IMG_DIM = 2048
HEADS = 4
DH = D // HEADS

NC, NS = 2, 16
EPT = E_INC // NS
ICH = 128
NCH = EPT // ICH
RNG = N_NODES // NC
NDUM = 16
RWT = RNG // NS
XCH = 120
BPT = RWT // LENS


def _make_pos():
    position = np.arange(LENS)[:, None].astype(np.float64)
    div_term = np.exp(np.arange(0, EMB, 2) * (-np.log(10000.0) / EMB))
    pe = np.zeros((LENS + 1, EMB))
    pe[1:, 0::2] = np.sin(position * div_term)
    pe[1:, 1::2] = np.cos(position * div_term)
    return np.ascontiguousarray((pe[:LENS] * 0.002).astype(np.float32))


_POS = _make_pos()


def _proj_body(ttab_ref, w1_ref, b1_ref, itab_ref, w2_ref, b2_ref,
               ot_ref, oi_ref):
    ot_ref[...] = jnp.dot(ttab_ref[...], w1_ref[...],
                          preferred_element_type=jnp.float32) + b1_ref[...]
    oi_ref[...] = jnp.dot(itab_ref[...], w2_ref[...],
                          preferred_element_type=jnp.float32) + b2_ref[...]


def _project_tables(ttab, w1, b1, itab, w2, b2):
    rows_blk = 400
    n = ttab.shape[0]

    def tspec(k):
        return pl.BlockSpec((k, EMB), lambda i: (0, 0))

    return pl.pallas_call(
        _proj_body,
        grid=(n // rows_blk,),
        in_specs=[
            pl.BlockSpec((rows_blk, TEXT_DIM), lambda i: (i, 0)),
            tspec(TEXT_DIM), pl.BlockSpec((1, EMB), lambda i: (0, 0)),
            pl.BlockSpec((rows_blk, IMG_DIM), lambda i: (i, 0)),
            tspec(IMG_DIM), pl.BlockSpec((1, EMB), lambda i: (0, 0)),
        ],
        out_specs=[pl.BlockSpec((rows_blk, EMB), lambda i: (i, 0))] * 2,
        out_shape=[jax.ShapeDtypeStruct((n, EMB), jnp.float32)] * 2,
    )(ttab, w1, b1.reshape(1, EMB), itab, w2, b2.reshape(1, EMB))


_MMBLK = 1200


def _mm2_pos_body(xt_ref, tht_ref, xi_ref, thi_ref, pos_ref, yt_ref, yi_ref):
    pos4 = jnp.concatenate([pos_ref[...]] * (_MMBLK // LENS), axis=0)
    yt_ref[...] = jnp.dot(xt_ref[...] + pos4, tht_ref[...],
                          preferred_element_type=jnp.float32)
    yi_ref[...] = jnp.dot(xi_ref[...] + pos4, thi_ref[...],
                          preferred_element_type=jnp.float32)


def _mm2_scaled_body(xt_ref, tht_ref, xi_ref, thi_ref, d_ref,
                     bt_ref, bi_ref, yt_ref, yi_ref):
    d = d_ref[...]
    xt = xt_ref[...] * d + bt_ref[...]
    xi = xi_ref[...] * d + bi_ref[...]
    yt_ref[...] = jnp.dot(xt, tht_ref[...], preferred_element_type=jnp.float32)
    yi_ref[...] = jnp.dot(xi, thi_ref[...], preferred_element_type=jnp.float32)


def _layer_matmul(xt, tht, xi, thi, pos=None, dinv=None, bt=None, bi=None):
    xspec = pl.BlockSpec((_MMBLK, EMB), lambda i: (i, 0))
    tspec = pl.BlockSpec((EMB, EMB), lambda i: (0, 0))
    bspec = pl.BlockSpec((1, EMB), lambda i: (0, 0))
    dspec = pl.BlockSpec((_MMBLK, 1), lambda i: (i, 0))
    pspec = pl.BlockSpec((LENS, EMB), lambda i: (0, 0))
    out_shape = [jax.ShapeDtypeStruct((N_NODES, EMB), jnp.float32)] * 2
    if dinv is None:
        return pl.pallas_call(
            _mm2_pos_body,
            grid=(N_NODES // _MMBLK,),
            in_specs=[xspec, tspec, xspec, tspec, pspec],
            out_specs=[xspec, xspec],
            out_shape=out_shape,
        )(xt, tht, xi, thi, pos)
    return pl.pallas_call(
        _mm2_scaled_body,
        grid=(N_NODES // _MMBLK,),
        in_specs=[xspec, tspec, xspec, tspec, dspec, bspec, bspec],
        out_specs=[xspec, xspec],
        out_shape=out_shape,
    )(xt, tht, xi, thi, dinv.reshape(N_NODES, 1),
      bt.reshape(1, EMB), bi.reshape(1, EMB))


def _escale_body(et_ref, ei_ref, b_ref, ot_ref, oi_ref):
    b = b_ref[...]
    ot_ref[...] = et_ref[...] * b
    oi_ref[...] = ei_ref[...] * b


def _escale(et, ei, binv):
    xspec = pl.BlockSpec((_MMBLK, EMB), lambda i: (i, 0))
    dspec = pl.BlockSpec((_MMBLK, 1), lambda i: (i, 0))
    return pl.pallas_call(
        _escale_body,
        grid=(N_NODES // _MMBLK,),
        in_specs=[xspec, xspec, dspec],
        out_specs=[xspec, xspec],
        out_shape=[jax.ShapeDtypeStruct((N_NODES, EMB), jnp.float32)] * 2,
    )(et, ei, binv.reshape(N_NODES, 1))


def _head_body(tg0_ref, ig0_ref, d0_ref, bt1_ref, bi1_ref,
               user_ref, w3_ref, b3_ref,
               wq_ref, bq_ref, wk_ref, bk_ref, wv_ref, bv_ref,
               wo_ref, bo_ref, wf1_ref, bf1_ref, wf2_ref, bf2_ref,
               ln1g_ref, ln1b_ref, ln2g_ref, ln2b_ref,
               dw_ref, db_ref, w4_ref, b4_ref, out_ref):
    f32 = jnp.float32

    def mm(a, b):
        return jnp.dot(a, b, preferred_element_type=f32)

    def layer_norm(x, g, b):
        m = jnp.mean(x, axis=-1, keepdims=True)
        v = jnp.mean((x - m) * (x - m), axis=-1, keepdims=True)
        return (x - m) / jnp.sqrt(v + 1e-5) * g + b

    d0 = d0_ref[...]
    tg0 = tg0_ref[...] * d0 + bt1_ref[...]
    ig0 = ig0_ref[...] * d0 + bi1_ref[...]
    user = user_ref[...][:, :HALF]
    text_user = jnp.concatenate([tg0, user], axis=1)
    img_user = jnp.concatenate([ig0, user], axis=1)
    tiu = mm(jnp.concatenate([text_user, img_user], axis=1),
             w3_ref[...]) + b3_ref[...]
    q = mm(tiu, wq_ref[...]) + bq_ref[...]
    k1 = mm(text_user, wk_ref[...]) + bk_ref[...]
    k2 = mm(img_user, wk_ref[...]) + bk_ref[...]
    v1 = mm(text_user, wv_ref[...]) + bv_ref[...]
    v2 = mm(img_user, wv_ref[...]) + bv_ref[...]
    scale = np.float32(1.0 / np.sqrt(DH))
    ao_parts = []
    for h in range(HEADS):
        sl = slice(h * DH, (h + 1) * DH)
        qh, k1h, k2h = q[:, sl], k1[:, sl], k2[:, sl]
        s1 = jnp.sum(qh * k1h, axis=1, keepdims=True) * scale
        s2 = jnp.sum(qh * k2h, axis=1, keepdims=True) * scale
        m = jnp.maximum(s1, s2)
        e1 = jnp.exp(s1 - m)
        e2 = jnp.exp(s2 - m)
        tot = e1 + e2
        ao_parts.append((e1 / tot) * v1[:, sl] + (e2 / tot) * v2[:, sl])
    ao = mm(jnp.concatenate(ao_parts, axis=1), wo_ref[...]) + bo_ref[...]
    x = layer_norm(tiu + ao, ln1g_ref[...], ln1b_ref[...])
    ff = mm(jnp.maximum(mm(x, wf1_ref[...]) + bf1_ref[...], 0.0),
            wf2_ref[...]) + bf2_ref[...]
    x = layer_norm(x + ff, ln2g_ref[...], ln2b_ref[...])
    x = jnp.maximum(mm(x, dw_ref[...]) + db_ref[...], 0.0)
    out_ref[...] = mm(x, w4_ref[...]) + b4_ref[...]


def _head(tg0, ig0, d0, bt1, bi1, user, p):
    w4p = jnp.pad(p['W4'], ((0, 0), (0, 127)))
    b4p = jnp.pad(p['b4'], (0, 127)).reshape(1, 128)
    args = [tg0, ig0, d0, bt1.reshape(1, EMB), bi1.reshape(1, EMB), user,
            p['W3'], p['b3'].reshape(1, D),
            p['Wq'], p['bq'].reshape(1, D), p['Wk'], p['bk'].reshape(1, D),
            p['Wv'], p['bv'].reshape(1, D), p['Wo'], p['bo'].reshape(1, D),
            p['Wf1'], p['bf1'].reshape(1, D), p['Wf2'], p['bf2'].reshape(1, D),
            p['ln1_g'].reshape(1, D), p['ln1_b'].reshape(1, D),
            p['ln2_g'].reshape(1, D), p['ln2_b'].reshape(1, D),
            p['dW'], p['db'].reshape(1, D), w4p, b4p]
    out = pl.pallas_call(
        _head_body,
        out_shape=jax.ShapeDtypeStruct((BSZ, 128), jnp.float32),
    )(*args)
    return out[:, :1]


_MESH = plsc.VectorSubcoreMesh(
    core_axis_name="c", subcore_axis_name="s", num_cores=NC, num_subcores=NS)


_PADK = 0x7FFF


@functools.partial(
    pl.kernel,
    out_type=[
        jax.ShapeDtypeStruct((N_NODES, EMB), jnp.float32),
        jax.ShapeDtypeStruct((N_NODES, EMB), jnp.float32),
        jax.ShapeDtypeStruct((BSZ, EMB), jnp.float32),
        jax.ShapeDtypeStruct((N_NODES,), jnp.float32),
        jax.ShapeDtypeStruct((N_NODES,), jnp.float32),
        jax.ShapeDtypeStruct((NC, NS, EPT), jnp.int32),
        jax.ShapeDtypeStruct((NC, NS, 16), jnp.int32),
        jax.ShapeDtypeStruct((NC, NS, EPT), jnp.int32),
        jax.ShapeDtypeStruct((NC, NS, 16), jnp.int32),
    ],
    mesh=_MESH,
    compiler_params=pltpu.CompilerParams(needs_layout_passes=False),
    scratch_types=[
        pltpu.VMEM((XCH,), jnp.int32),
        pltpu.VMEM((XCH, EMB), jnp.float32),
        pltpu.VMEM_SHARED((N_NODES,), jnp.float32),
        pltpu.VMEM((RWT * 2,), jnp.float32),
        pltpu.VMEM((ICH,), jnp.float32),
        pltpu.VMEM((ICH,), jnp.int32),
        pltpu.VMEM((ICH,), jnp.int32),
        pltpu.VMEM((ICH,), jnp.int32),
        pltpu.VMEM((ICH,), jnp.int32),
        pltpu.VMEM((EPT,), jnp.int32),
        pltpu.VMEM((EPT,), jnp.int32),
        pltpu.VMEM((16,), jnp.int32),
        pltpu.VMEM((8,), jnp.int32),
        pltpu.VMEM((8, EMB), jnp.float32),
        pltpu.SemaphoreType.DMA,
        pltpu.SemaphoreType.DMA,
        pltpu.SemaphoreType.DMA,
        pltpu.SemaphoreType.DMA,
        pltpu.SemaphoreType.DMA,
    ],
)
def _sc_prep(pt_hbm, pi_hbm, ri_hbm, node_hbm, edge_hbm, uid_hbm, ut_hbm,
             xt_hbm, xi_hbm, user_hbm, dinv_hbm, binv_hbm,
             pke_hbm, cnte_hbm, pkn_hbm, cntn_hbm,
             idxv, rows, deg, degv, onesv, nbvA, ebvA, nbvB, ebvB,
             pebuf, pnbuf, cntv, uidv, urows, sem, isA, isB, dsA, dsB):
    c = lax.axis_index("c")
    s = lax.axis_index("s")
    w = c * NS + s
    nb = w * (N_NODES // (NC * NS))
    dpt = RWT * 2
    lo = c * RNG
    hi = lo + RNG

    def zdeg(j, _):
        degv[pl.ds(j * 16, 16)] = jnp.zeros((16,), jnp.float32)
        return 0
    lax.fori_loop(0, dpt // 16, zdeg, 0)
    pltpu.sync_copy(degv, deg.at[pl.ds(s * dpt, dpt)])
    for j in range(ICH // 16):
        onesv[pl.ds(j * 16, 16)] = jnp.ones((16,), jnp.float32)

    lane = lax.iota(jnp.int32, 16)
    pad_e = lane | (_PADK << 15)
    pad_n = _PADK | (lane << 15)

    def zpk(j, _):
        sl = pl.ds(j * 16, 16)
        pebuf[sl] = pad_e
        pnbuf[sl] = pad_n
        return 0
    lax.fori_loop(0, EPT // 16, zpk, 0)
    plsc.subcore_barrier()

    pbufs = ((nbvA, ebvA, isA, dsA), (nbvB, ebvB, isB, dsB))
    pltpu.sync_copy(node_hbm.at[pl.ds(s * EPT, ICH)], nbvA)
    pltpu.sync_copy(edge_hbm.at[pl.ds(s * EPT, ICH)], ebvA)
    pltpu.async_copy(node_hbm.at[pl.ds(s * EPT + ICH, ICH)], nbvB, isB)
    pltpu.async_copy(edge_hbm.at[pl.ds(s * EPT + ICH, ICH)], ebvB, isB)

    def part_body(q, carry):
        ce, cn = carry
        for x, (nbv, ebv, isem, dsem) in enumerate(pbufs):
            cx = 2 * q + x

            def wait_idx():
                pltpu.make_async_copy(
                    node_hbm.at[pl.ds(0, ICH)], nbv, isem).wait()
                pltpu.make_async_copy(
                    edge_hbm.at[pl.ds(0, ICH)], ebv, isem).wait()

            if x == 0:
                @pl.when(q >= 1)
                def _():
                    wait_idx()
            else:
                wait_idx()

            @pl.when(c == 0)
            def _():
                pltpu.async_copy(onesv, deg.at[nbv], dsem, add=True)

            @pl.when(c == 1)
            def _():
                pltpu.async_copy(onesv, deg.at[ebv], dsem, add=True)

            one = jnp.ones((16,), jnp.int32)
            zero = jnp.zeros((16,), jnp.int32)
            for j in range(ICH // 16):
                sl = pl.ds(j * 16, 16)
                nv = nbv[sl]
                ev = ebv[sl]
                pk = nv | (ev << 15)
                me = (ev >= lo) & (ev < hi)
                cse = plsc.cumsum(jnp.where(me, one, zero))
                plsc.store_scatter(pebuf, [ce + cse - 1], pk, mask=me)
                ce = ce + jnp.max(cse)
                mn = (nv >= lo) & (nv < hi)
                csn = plsc.cumsum(jnp.where(mn, one, zero))
                plsc.store_scatter(pnbuf, [cn + csn - 1], pk, mask=mn)
                cn = cn + jnp.max(csn)

            @pl.when(c == 0)
            def _():
                pltpu.make_async_copy(onesv, deg.at[nbv], dsem).wait()

            @pl.when(c == 1)
            def _():
                pltpu.make_async_copy(onesv, deg.at[ebv], dsem).wait()

            @pl.when(cx + 2 < NCH)
            def _():
                base = s * EPT + (cx + 2) * ICH
                pltpu.async_copy(node_hbm.at[pl.ds(base, ICH)], nbv, isem)
                pltpu.async_copy(edge_hbm.at[pl.ds(base, ICH)], ebv, isem)
        return ce, cn

    ce, cn = lax.fori_loop(0, NCH // 2, part_body,
                           (jnp.int32(0), jnp.int32(0)))

    qe = jnp.maximum(jnp.minimum((ce + 383) // 384 * 384, EPT) // 384, 1)
    qn = jnp.maximum(jnp.minimum((cn + 383) // 384 * 384, EPT) // 384, 1)
    cntv[pl.ds(0, 16)] = jnp.full((16,), qe, jnp.int32)
    pltpu.sync_copy(cntv, cnte_hbm.at[c, s])
    pltpu.sync_copy(pebuf, pke_hbm.at[c, s])
    cntv[pl.ds(0, 16)] = jnp.full((16,), qn, jnp.int32)
    pltpu.sync_copy(cntv, cntn_hbm.at[c, s])
    pltpu.sync_copy(pnbuf, pkn_hbm.at[c, s])

    for p_hbm, x_hbm in ((pt_hbm, xt_hbm), (pi_hbm, xi_hbm)):
        def xbody(k, _):
            pltpu.sync_copy(ri_hbm.at[pl.ds(nb + k * XCH, XCH)], idxv)
            pltpu.async_copy(p_hbm.at[idxv], rows, sem).wait()
            pltpu.sync_copy(rows, x_hbm.at[pl.ds(nb + k * XCH, XCH)])
            return 0
        lax.fori_loop(0, (N_NODES // (NC * NS)) // XCH, xbody, 0)

    @pl.when((c == 0) & (s < 8))
    def _():
        pltpu.sync_copy(uid_hbm.at[pl.ds(s * 8, 8)], uidv)
        pltpu.async_copy(ut_hbm.at[uidv], urows, sem).wait()
        pltpu.sync_copy(urows, user_hbm.at[pl.ds(s * 8, 8)])

    plsc.subcore_barrier()

    pltpu.sync_copy(deg.at[pl.ds(s * dpt, dpt)], degv)

    def inv(j, _):
        sl = pl.ds(j * 16, 16)
        v = degv[sl]
        degv[sl] = jnp.where(v > 0.0, 1.0 / v, 0.0)
        return 0
    lax.fori_loop(0, dpt // 16, inv, 0)

    @pl.when(c == 0)
    def _():
        pltpu.sync_copy(degv, dinv_hbm.at[pl.ds(s * dpt, dpt)])

    @pl.when(c == 1)
    def _():
        pltpu.sync_copy(degv, binv_hbm.at[pl.ds(s * dpt, dpt)])


def _make_seg(key_low):
    @functools.partial(
        pl.kernel,
        out_type=[
            jax.ShapeDtypeStruct((N_NODES, EMB), jnp.float32),
            jax.ShapeDtypeStruct((N_NODES, EMB), jnp.float32),
        ],
        mesh=_MESH,
        compiler_params=pltpu.CompilerParams(needs_layout_passes=False),
        scratch_types=[
            pltpu.VMEM_SHARED((RNG + NDUM, EMB), jnp.float32),
            pltpu.VMEM((ICH,), jnp.int32),
            pltpu.VMEM((ICH,), jnp.int32),
            pltpu.VMEM((ICH,), jnp.int32),
            pltpu.VMEM((ICH,), jnp.int32),
            pltpu.VMEM((ICH,), jnp.int32),
            pltpu.VMEM((ICH,), jnp.int32),
            pltpu.VMEM((ICH, EMB), jnp.float32),
            pltpu.VMEM((ICH, EMB), jnp.float32),
            pltpu.VMEM((ICH, EMB), jnp.float32),
            pltpu.VMEM((24, EMB), jnp.float32),
            pltpu.VMEM((16,), jnp.int32),
            pltpu.SemaphoreType.DMA,
            pltpu.SemaphoreType.DMA,
            pltpu.SemaphoreType.DMA,
            pltpu.SemaphoreType.DMA,
            pltpu.SemaphoreType.DMA,
        ],
    )
    def seg(gt_hbm, gi_hbm, pk_hbm, cnt_hbm, ot_hbm, oi_hbm,
            acc, nvA, nvB, nvC, evcA, evcB, evcC, rowsA, rowsB, rowsC,
            zbuf, cbuf, gsA, gsB, gsC, ss, zs):
        c = lax.axis_index("c")
        s = lax.axis_index("s")
        lo = c * RNG
        r0 = s * RWT
        bufs = ((nvA, evcA, rowsA, gsA),
                (nvB, evcB, rowsB, gsB),
                (nvC, evcC, rowsC, gsC))
        nbuf = len(bufs)

        def zb(r, _):
            for j in range(EMB // 16):
                zbuf[r, pl.ds(j * 16, 16)] = jnp.zeros((16,), jnp.float32)
            return 0
        lax.fori_loop(0, 24, zb, 0)

        pltpu.sync_copy(cnt_hbm.at[c, s], cbuf)
        qtrips = jnp.max(cbuf[pl.ds(0, 16)])

        def load_idx_and_gather(cx, nv, evc, rows, gs, g_hbm):
            pltpu.sync_copy(pk_hbm.at[c, s, pl.ds(cx * ICH, ICH)], nv)
            for j in range(ICH // 16):
                sl = pl.ds(j * 16, 16)
                v = nv[sl]
                if key_low:
                    g, e = v >> 15, v & _PADK
                else:
                    g, e = v & _PADK, v >> 15
                loc = e - lo
                dummy = RNG + (g & (NDUM - 1))
                ok = (loc >= 0) & (loc < RNG)
                evc[sl] = jnp.where(ok, loc, dummy)
                nv[sl] = g
            pltpu.async_copy(g_hbm.at[nv], rows, gs)

        for m, (g_hbm, o_hbm) in enumerate(((gt_hbm, ot_hbm),
                                            (gi_hbm, oi_hbm))):
            for k in range(RWT // 24):
                pltpu.async_copy(zbuf, acc.at[pl.ds(r0 + k * 24, 24)], zs)
            for k in range(RWT // 24):
                pltpu.make_async_copy(
                    zbuf, acc.at[pl.ds(r0 + k * 24, 24)], zs).wait()
            plsc.subcore_barrier()

            load_idx_and_gather(0, *bufs[0], g_hbm)
            load_idx_and_gather(1, *bufs[1], g_hbm)

            def body(q, _):
                for x in range(nbuf):
                    cx = nbuf * q + x
                    nv, evc, rows, gs = bufs[x]
                    nv2, evc2, rows2, gs2 = bufs[(x + 2) % nbuf]
                    if x == 0:
                        @pl.when(q >= 1)
                        def _():
                            pltpu.make_async_copy(
                                rows2, acc.at[evc2], ss).wait()
                    else:
                        pltpu.make_async_copy(
                            bufs[x - 1][2], acc.at[bufs[x - 1][1]],
                            ss).wait()

                    @pl.when(cx + 2 < nbuf * qtrips)
                    def _():
                        load_idx_and_gather(cx + 2, nv2, evc2, rows2, gs2,
                                            g_hbm)
                    pltpu.make_async_copy(g_hbm.at[nv], rows, gs).wait()
                    pltpu.async_copy(rows, acc.at[evc], ss, add=True)
                return 0
            lax.fori_loop(0, qtrips, body, 0)
            pltpu.make_async_copy(rowsC, acc.at[evcC], ss).wait()
            plsc.subcore_barrier()

            pltpu.sync_copy(acc.at[pl.ds(r0, RWT)],
                            o_hbm.at[pl.ds(lo + r0, RWT)])
            if m == 0:
                plsc.subcore_barrier()

    return seg


_seg_s1 = _make_seg(False)
_seg_s2 = _make_seg(True)


def kernel(input, hg_idx, related_items, label, uid, params):
    p = params
    node = hg_idx[0]
    edge = hg_idx[1]

    pt, pi = _project_tables(p['text_table'], p['W1'], p['b1'],
                             p['img_table'], p['W2'], p['b2'])

    pos = jnp.asarray(_POS)
    ut_p = jnp.pad(p['user_table'], ((0, 0), (0, EMB - HALF)))
    (xt, xi, user, dinv, binv,
     pk_e, cnt_e, pk_n, cnt_n) = _sc_prep(
        pt, pi, related_items, node, edge, uid, ut_p)

    yt, yi = _layer_matmul(xt, p['theta_t0'], xi, p['theta_i0'], pos=pos)
    et_raw, ei_raw = _seg_s1(yt, yi, pk_e, cnt_e)
    et, ei = _escale(et_raw, ei_raw, binv)
    xt2, xi2 = _seg_s2(et, ei, pk_n, cnt_n)

    yt2, yi2 = _layer_matmul(xt2, p['theta_t1'], xi2, p['theta_i1'],
                             dinv=dinv, bt=p['bias_t0'], bi=p['bias_i0'])
    et2_raw, ei2_raw = _seg_s1(yt2, yi2, pk_e, cnt_e)
    et2, ei2 = _escale(et2_raw, ei2_raw, binv)
    xt3, xi3 = _seg_s2(et2, ei2, pk_n, cnt_n)
    tg0_raw = xt3[::LENS]
    ig0_raw = xi3[::LENS]

    d0 = dinv[::LENS].reshape(BSZ, 1)
    return _head(tg0_raw, ig0_raw, d0, p['bias_t1'], p['bias_i1'], user, p)


# speedup vs baseline: 10.6745x; 1.1381x over previous
import functools

import jax
import jax.numpy as jnp
import numpy as np
from jax import lax
from jax.experimental import pallas as pl
from jax.experimental.pallas import tpu as pltpu
from jax.experimental.pallas import tpu_sc as plsc

BSZ = 64
LENS = 300
EMB = 128
HALF = EMB // 2
D = EMB + HALF
N_NODES = BSZ * LENS
E_INC = 307200
N_ITEMS = 20000
TEXT_DIM = 384
IMG_DIM = 2048
HEADS = 4
DH = D // HEADS

NC, NS = 2, 16
EPT = E_INC // NS
ICH = 128
NCH = EPT // ICH
RNG = N_NODES // NC
NDUM = 16
RWT = RNG // NS
XCH = 120
BPT = RWT // LENS


def _make_pos():
    position = np.arange(LENS)[:, None].astype(np.float64)
    div_term = np.exp(np.arange(0, EMB, 2) * (-np.log(10000.0) / EMB))
    pe = np.zeros((LENS + 1, EMB))
    pe[1:, 0::2] = np.sin(position * div_term)
    pe[1:, 1::2] = np.cos(position * div_term)
    return np.ascontiguousarray((pe[:LENS] * 0.002).astype(np.float32))


_POS = _make_pos()


def _proj_body(ttab_ref, w1_ref, b1_ref, itab_ref, w2_ref, b2_ref,
               ot_ref, oi_ref):
    ot_ref[...] = jnp.dot(ttab_ref[...], w1_ref[...],
                          preferred_element_type=jnp.float32) + b1_ref[...]
    oi_ref[...] = jnp.dot(itab_ref[...], w2_ref[...],
                          preferred_element_type=jnp.float32) + b2_ref[...]


def _project_tables(ttab, w1, b1, itab, w2, b2):
    rows_blk = 400
    n = ttab.shape[0]

    def tspec(k):
        return pl.BlockSpec((k, EMB), lambda i: (0, 0))

    return pl.pallas_call(
        _proj_body,
        grid=(n // rows_blk,),
        in_specs=[
            pl.BlockSpec((rows_blk, TEXT_DIM), lambda i: (i, 0)),
            tspec(TEXT_DIM), pl.BlockSpec((1, EMB), lambda i: (0, 0)),
            pl.BlockSpec((rows_blk, IMG_DIM), lambda i: (i, 0)),
            tspec(IMG_DIM), pl.BlockSpec((1, EMB), lambda i: (0, 0)),
        ],
        out_specs=[pl.BlockSpec((rows_blk, EMB), lambda i: (i, 0))] * 2,
        out_shape=[jax.ShapeDtypeStruct((n, EMB), jnp.float32)] * 2,
    )(ttab, w1, b1.reshape(1, EMB), itab, w2, b2.reshape(1, EMB))


_MMBLK = 1200


def _mm2_pos_body(xt_ref, tht_ref, xi_ref, thi_ref, pos_ref, yt_ref, yi_ref):
    pos4 = jnp.concatenate([pos_ref[...]] * (_MMBLK // LENS), axis=0)
    yt_ref[...] = jnp.dot(xt_ref[...] + pos4, tht_ref[...],
                          preferred_element_type=jnp.float32)
    yi_ref[...] = jnp.dot(xi_ref[...] + pos4, thi_ref[...],
                          preferred_element_type=jnp.float32)


def _mm2_scaled_body(xt_ref, tht_ref, xi_ref, thi_ref, d_ref,
                     bt_ref, bi_ref, yt_ref, yi_ref):
    d = d_ref[...]
    xt = xt_ref[...] * d + bt_ref[...]
    xi = xi_ref[...] * d + bi_ref[...]
    yt_ref[...] = jnp.dot(xt, tht_ref[...], preferred_element_type=jnp.float32)
    yi_ref[...] = jnp.dot(xi, thi_ref[...], preferred_element_type=jnp.float32)


def _layer_matmul(xt, tht, xi, thi, pos=None, dinv=None, bt=None, bi=None):
    xspec = pl.BlockSpec((_MMBLK, EMB), lambda i: (i, 0))
    tspec = pl.BlockSpec((EMB, EMB), lambda i: (0, 0))
    bspec = pl.BlockSpec((1, EMB), lambda i: (0, 0))
    dspec = pl.BlockSpec((_MMBLK, 1), lambda i: (i, 0))
    pspec = pl.BlockSpec((LENS, EMB), lambda i: (0, 0))
    out_shape = [jax.ShapeDtypeStruct((N_NODES, EMB), jnp.float32)] * 2
    if dinv is None:
        return pl.pallas_call(
            _mm2_pos_body,
            grid=(N_NODES // _MMBLK,),
            in_specs=[xspec, tspec, xspec, tspec, pspec],
            out_specs=[xspec, xspec],
            out_shape=out_shape,
        )(xt, tht, xi, thi, pos)
    return pl.pallas_call(
        _mm2_scaled_body,
        grid=(N_NODES // _MMBLK,),
        in_specs=[xspec, tspec, xspec, tspec, dspec, bspec, bspec],
        out_specs=[xspec, xspec],
        out_shape=out_shape,
    )(xt, tht, xi, thi, dinv.reshape(N_NODES, 1),
      bt.reshape(1, EMB), bi.reshape(1, EMB))


def _escale_body(et_ref, ei_ref, b_ref, ot_ref, oi_ref):
    b = b_ref[...]
    ot_ref[...] = et_ref[...] * b
    oi_ref[...] = ei_ref[...] * b


def _escale(et, ei, binv):
    xspec = pl.BlockSpec((_MMBLK, EMB), lambda i: (i, 0))
    dspec = pl.BlockSpec((_MMBLK, 1), lambda i: (i, 0))
    return pl.pallas_call(
        _escale_body,
        grid=(N_NODES // _MMBLK,),
        in_specs=[xspec, xspec, dspec],
        out_specs=[xspec, xspec],
        out_shape=[jax.ShapeDtypeStruct((N_NODES, EMB), jnp.float32)] * 2,
    )(et, ei, binv.reshape(N_NODES, 1))


def _head_body(tg0_ref, ig0_ref, d0_ref, bt1_ref, bi1_ref,
               user_ref, w3_ref, b3_ref,
               wq_ref, bq_ref, wk_ref, bk_ref, wv_ref, bv_ref,
               wo_ref, bo_ref, wf1_ref, bf1_ref, wf2_ref, bf2_ref,
               ln1g_ref, ln1b_ref, ln2g_ref, ln2b_ref,
               dw_ref, db_ref, w4_ref, b4_ref, out_ref):
    f32 = jnp.float32

    def mm(a, b):
        return jnp.dot(a, b, preferred_element_type=f32)

    def layer_norm(x, g, b):
        m = jnp.mean(x, axis=-1, keepdims=True)
        v = jnp.mean((x - m) * (x - m), axis=-1, keepdims=True)
        return (x - m) / jnp.sqrt(v + 1e-5) * g + b

    d0 = d0_ref[...]
    tg0 = tg0_ref[...] * d0 + bt1_ref[...]
    ig0 = ig0_ref[...] * d0 + bi1_ref[...]
    user = user_ref[...][:, :HALF]
    text_user = jnp.concatenate([tg0, user], axis=1)
    img_user = jnp.concatenate([ig0, user], axis=1)
    tiu = mm(jnp.concatenate([text_user, img_user], axis=1),
             w3_ref[...]) + b3_ref[...]
    q = mm(tiu, wq_ref[...]) + bq_ref[...]
    k1 = mm(text_user, wk_ref[...]) + bk_ref[...]
    k2 = mm(img_user, wk_ref[...]) + bk_ref[...]
    v1 = mm(text_user, wv_ref[...]) + bv_ref[...]
    v2 = mm(img_user, wv_ref[...]) + bv_ref[...]
    scale = np.float32(1.0 / np.sqrt(DH))
    ao_parts = []
    for h in range(HEADS):
        sl = slice(h * DH, (h + 1) * DH)
        qh, k1h, k2h = q[:, sl], k1[:, sl], k2[:, sl]
        s1 = jnp.sum(qh * k1h, axis=1, keepdims=True) * scale
        s2 = jnp.sum(qh * k2h, axis=1, keepdims=True) * scale
        m = jnp.maximum(s1, s2)
        e1 = jnp.exp(s1 - m)
        e2 = jnp.exp(s2 - m)
        tot = e1 + e2
        ao_parts.append((e1 / tot) * v1[:, sl] + (e2 / tot) * v2[:, sl])
    ao = mm(jnp.concatenate(ao_parts, axis=1), wo_ref[...]) + bo_ref[...]
    x = layer_norm(tiu + ao, ln1g_ref[...], ln1b_ref[...])
    ff = mm(jnp.maximum(mm(x, wf1_ref[...]) + bf1_ref[...], 0.0),
            wf2_ref[...]) + bf2_ref[...]
    x = layer_norm(x + ff, ln2g_ref[...], ln2b_ref[...])
    x = jnp.maximum(mm(x, dw_ref[...]) + db_ref[...], 0.0)
    out_ref[...] = mm(x, w4_ref[...]) + b4_ref[...]


def _head(tg0, ig0, d0, bt1, bi1, user, p):
    w4p = jnp.pad(p['W4'], ((0, 0), (0, 127)))
    b4p = jnp.pad(p['b4'], (0, 127)).reshape(1, 128)
    args = [tg0, ig0, d0, bt1.reshape(1, EMB), bi1.reshape(1, EMB), user,
            p['W3'], p['b3'].reshape(1, D),
            p['Wq'], p['bq'].reshape(1, D), p['Wk'], p['bk'].reshape(1, D),
            p['Wv'], p['bv'].reshape(1, D), p['Wo'], p['bo'].reshape(1, D),
            p['Wf1'], p['bf1'].reshape(1, D), p['Wf2'], p['bf2'].reshape(1, D),
            p['ln1_g'].reshape(1, D), p['ln1_b'].reshape(1, D),
            p['ln2_g'].reshape(1, D), p['ln2_b'].reshape(1, D),
            p['dW'], p['db'].reshape(1, D), w4p, b4p]
    out = pl.pallas_call(
        _head_body,
        out_shape=jax.ShapeDtypeStruct((BSZ, 128), jnp.float32),
    )(*args)
    return out[:, :1]


_MESH = plsc.VectorSubcoreMesh(
    core_axis_name="c", subcore_axis_name="s", num_cores=NC, num_subcores=NS)


_PADK = 0x7FFF


@functools.partial(
    pl.kernel,
    out_type=[
        jax.ShapeDtypeStruct((N_NODES, EMB), jnp.float32),
        jax.ShapeDtypeStruct((N_NODES, EMB), jnp.float32),
        jax.ShapeDtypeStruct((BSZ, EMB), jnp.float32),
        jax.ShapeDtypeStruct((N_NODES,), jnp.float32),
        jax.ShapeDtypeStruct((N_NODES,), jnp.float32),
        jax.ShapeDtypeStruct((NC, NS, EPT), jnp.int32),
        jax.ShapeDtypeStruct((NC, NS, 16), jnp.int32),
        jax.ShapeDtypeStruct((NC, NS, EPT), jnp.int32),
        jax.ShapeDtypeStruct((NC, NS, 16), jnp.int32),
    ],
    mesh=_MESH,
    compiler_params=pltpu.CompilerParams(needs_layout_passes=False),
    scratch_types=[
        pltpu.VMEM((XCH,), jnp.int32),
        pltpu.VMEM((XCH, EMB), jnp.float32),
        pltpu.VMEM_SHARED((N_NODES,), jnp.float32),
        pltpu.VMEM((RWT * 2,), jnp.float32),
        pltpu.VMEM((ICH,), jnp.float32),
        pltpu.VMEM((ICH,), jnp.int32),
        pltpu.VMEM((ICH,), jnp.int32),
        pltpu.VMEM((ICH,), jnp.int32),
        pltpu.VMEM((ICH,), jnp.int32),
        pltpu.VMEM((EPT,), jnp.int32),
        pltpu.VMEM((EPT,), jnp.int32),
        pltpu.VMEM((16,), jnp.int32),
        pltpu.VMEM((8,), jnp.int32),
        pltpu.VMEM((8, EMB), jnp.float32),
        pltpu.SemaphoreType.DMA,
        pltpu.SemaphoreType.DMA,
        pltpu.SemaphoreType.DMA,
        pltpu.SemaphoreType.DMA,
        pltpu.SemaphoreType.DMA,
    ],
)
def _sc_prep(pt_hbm, pi_hbm, ri_hbm, node_hbm, edge_hbm, uid_hbm, ut_hbm,
             xt_hbm, xi_hbm, user_hbm, dinv_hbm, binv_hbm,
             pke_hbm, cnte_hbm, pkn_hbm, cntn_hbm,
             idxv, rows, deg, degv, onesv, nbvA, ebvA, nbvB, ebvB,
             pebuf, pnbuf, cntv, uidv, urows, sem, isA, isB, dsA, dsB):
    c = lax.axis_index("c")
    s = lax.axis_index("s")
    w = c * NS + s
    nb = w * (N_NODES // (NC * NS))
    dpt = RWT * 2
    lo = c * RNG
    hi = lo + RNG

    def zdeg(j, _):
        degv[pl.ds(j * 16, 16)] = jnp.zeros((16,), jnp.float32)
        return 0
    lax.fori_loop(0, dpt // 16, zdeg, 0)
    pltpu.sync_copy(degv, deg.at[pl.ds(s * dpt, dpt)])
    for j in range(ICH // 16):
        onesv[pl.ds(j * 16, 16)] = jnp.ones((16,), jnp.float32)

    lane = lax.iota(jnp.int32, 16)
    pad_e = lane | (_PADK << 15)
    pad_n = _PADK | (lane << 15)

    def zpk(j, _):
        sl = pl.ds(j * 16, 16)
        pebuf[sl] = pad_e
        pnbuf[sl] = pad_n
        return 0
    lax.fori_loop(0, EPT // 16, zpk, 0)
    plsc.subcore_barrier()

    pbufs = ((nbvA, ebvA, isA, dsA), (nbvB, ebvB, isB, dsB))
    pltpu.sync_copy(node_hbm.at[pl.ds(s * EPT, ICH)], nbvA)
    pltpu.sync_copy(edge_hbm.at[pl.ds(s * EPT, ICH)], ebvA)
    pltpu.async_copy(node_hbm.at[pl.ds(s * EPT + ICH, ICH)], nbvB, isB)
    pltpu.async_copy(edge_hbm.at[pl.ds(s * EPT + ICH, ICH)], ebvB, isB)

    def part_body(q, carry):
        ce, cn = carry
        for x, (nbv, ebv, isem, dsem) in enumerate(pbufs):
            cx = 2 * q + x

            def wait_idx():
                pltpu.make_async_copy(
                    node_hbm.at[pl.ds(0, ICH)], nbv, isem).wait()
                pltpu.make_async_copy(
                    edge_hbm.at[pl.ds(0, ICH)], ebv, isem).wait()

            if x == 0:
                @pl.when(q >= 1)
                def _():
                    wait_idx()
            else:
                wait_idx()

            @pl.when(c == 0)
            def _():
                pltpu.async_copy(onesv, deg.at[nbv], dsem, add=True)

            @pl.when(c == 1)
            def _():
                pltpu.async_copy(onesv, deg.at[ebv], dsem, add=True)

            one = jnp.ones((16,), jnp.int32)
            zero = jnp.zeros((16,), jnp.int32)
            for j in range(ICH // 16):
                sl = pl.ds(j * 16, 16)
                nv = nbv[sl]
                ev = ebv[sl]
                pk = nv | (ev << 15)
                me = (ev >= lo) & (ev < hi)
                cse = plsc.cumsum(jnp.where(me, one, zero))
                plsc.store_scatter(pebuf, [ce + cse - 1], pk, mask=me)
                ce = ce + jnp.max(cse)
                mn = (nv >= lo) & (nv < hi)
                csn = plsc.cumsum(jnp.where(mn, one, zero))
                plsc.store_scatter(pnbuf, [cn + csn - 1], pk, mask=mn)
                cn = cn + jnp.max(csn)

            @pl.when(c == 0)
            def _():
                pltpu.make_async_copy(onesv, deg.at[nbv], dsem).wait()

            @pl.when(c == 1)
            def _():
                pltpu.make_async_copy(onesv, deg.at[ebv], dsem).wait()

            @pl.when(cx + 2 < NCH)
            def _():
                base = s * EPT + (cx + 2) * ICH
                pltpu.async_copy(node_hbm.at[pl.ds(base, ICH)], nbv, isem)
                pltpu.async_copy(edge_hbm.at[pl.ds(base, ICH)], ebv, isem)
        return ce, cn

    ce, cn = lax.fori_loop(0, NCH // 2, part_body,
                           (jnp.int32(0), jnp.int32(0)))

    qe = jnp.maximum(jnp.minimum((ce + 383) // 384 * 384, EPT) // 384, 1)
    qn = jnp.maximum(jnp.minimum((cn + 383) // 384 * 384, EPT) // 384, 1)
    cntv[pl.ds(0, 16)] = jnp.full((16,), qe, jnp.int32)
    pltpu.sync_copy(cntv, cnte_hbm.at[c, s])
    pltpu.sync_copy(pebuf, pke_hbm.at[c, s])
    cntv[pl.ds(0, 16)] = jnp.full((16,), qn, jnp.int32)
    pltpu.sync_copy(cntv, cntn_hbm.at[c, s])
    pltpu.sync_copy(pnbuf, pkn_hbm.at[c, s])

    for p_hbm, x_hbm in ((pt_hbm, xt_hbm), (pi_hbm, xi_hbm)):
        def xbody(k, _):
            pltpu.sync_copy(ri_hbm.at[pl.ds(nb + k * XCH, XCH)], idxv)
            pltpu.async_copy(p_hbm.at[idxv], rows, sem).wait()
            pltpu.sync_copy(rows, x_hbm.at[pl.ds(nb + k * XCH, XCH)])
            return 0
        lax.fori_loop(0, (N_NODES // (NC * NS)) // XCH, xbody, 0)

    @pl.when((c == 0) & (s < 8))
    def _():
        pltpu.sync_copy(uid_hbm.at[pl.ds(s * 8, 8)], uidv)
        pltpu.async_copy(ut_hbm.at[uidv], urows, sem).wait()
        pltpu.sync_copy(urows, user_hbm.at[pl.ds(s * 8, 8)])

    plsc.subcore_barrier()

    pltpu.sync_copy(deg.at[pl.ds(s * dpt, dpt)], degv)

    def inv(j, _):
        sl = pl.ds(j * 16, 16)
        v = degv[sl]
        degv[sl] = jnp.where(v > 0.0, 1.0 / v, 0.0)
        return 0
    lax.fori_loop(0, dpt // 16, inv, 0)

    @pl.when(c == 0)
    def _():
        pltpu.sync_copy(degv, dinv_hbm.at[pl.ds(s * dpt, dpt)])

    @pl.when(c == 1)
    def _():
        pltpu.sync_copy(degv, binv_hbm.at[pl.ds(s * dpt, dpt)])


def _make_seg(key_low):
    @functools.partial(
        pl.kernel,
        out_type=[
            jax.ShapeDtypeStruct((N_NODES, EMB), jnp.float32),
            jax.ShapeDtypeStruct((N_NODES, EMB), jnp.float32),
        ],
        mesh=_MESH,
        compiler_params=pltpu.CompilerParams(needs_layout_passes=False),
        scratch_types=[
            pltpu.VMEM_SHARED((RNG + NDUM, EMB), jnp.float32),
            pltpu.VMEM((ICH,), jnp.int32),
            pltpu.VMEM((ICH,), jnp.int32),
            pltpu.VMEM((ICH,), jnp.int32),
            pltpu.VMEM((ICH,), jnp.int32),
            pltpu.VMEM((ICH,), jnp.int32),
            pltpu.VMEM((ICH,), jnp.int32),
            pltpu.VMEM((ICH, EMB), jnp.float32),
            pltpu.VMEM((ICH, EMB), jnp.float32),
            pltpu.VMEM((ICH, EMB), jnp.float32),
            pltpu.VMEM((24, EMB), jnp.float32),
            pltpu.VMEM((16,), jnp.int32),
            pltpu.SemaphoreType.DMA,
            pltpu.SemaphoreType.DMA,
            pltpu.SemaphoreType.DMA,
            pltpu.SemaphoreType.DMA,
            pltpu.SemaphoreType.DMA,
            pltpu.SemaphoreType.DMA,
            pltpu.SemaphoreType.DMA,
            pltpu.SemaphoreType.DMA,
        ],
    )
    def seg(gt_hbm, gi_hbm, pk_hbm, cnt_hbm, ot_hbm, oi_hbm,
            acc, nvA, nvB, nvC, evcA, evcB, evcC, rowsA, rowsB, rowsC,
            zbuf, cbuf, gsA, gsB, gsC, isA, isB, isC, ss, zs):
        c = lax.axis_index("c")
        s = lax.axis_index("s")
        lo = c * RNG
        r0 = s * RWT
        bufs = ((nvA, evcA, rowsA, gsA, isA),
                (nvB, evcB, rowsB, gsB, isB),
                (nvC, evcC, rowsC, gsC, isC))
        nbuf = len(bufs)

        def zb(r, _):
            for j in range(EMB // 16):
                zbuf[r, pl.ds(j * 16, 16)] = jnp.zeros((16,), jnp.float32)
            return 0
        lax.fori_loop(0, 24, zb, 0)

        pltpu.sync_copy(cnt_hbm.at[c, s], cbuf)
        qtrips = jnp.max(cbuf[pl.ds(0, 16)])

        def idx_load(cx, nv, isem):
            pltpu.async_copy(pk_hbm.at[c, s, pl.ds(cx * ICH, ICH)], nv,
                             isem)

        def idx_wait(nv, isem):
            pltpu.make_async_copy(pk_hbm.at[c, s, pl.ds(0, ICH)], nv,
                                  isem).wait()

        def decode_and_gather(nv, evc, rows, gs, g_hbm):
            for j in range(ICH // 16):
                sl = pl.ds(j * 16, 16)
                v = nv[sl]
                if key_low:
                    g, e = v >> 15, v & _PADK
                else:
                    g, e = v & _PADK, v >> 15
                loc = e - lo
                dummy = RNG + (g & (NDUM - 1))
                ok = (loc >= 0) & (loc < RNG)
                evc[sl] = jnp.where(ok, loc, dummy)
                nv[sl] = g
            pltpu.async_copy(g_hbm.at[nv], rows, gs)

        for m, (g_hbm, o_hbm) in enumerate(((gt_hbm, ot_hbm),
                                            (gi_hbm, oi_hbm))):
            for k in range(RWT // 24):
                pltpu.async_copy(zbuf, acc.at[pl.ds(r0 + k * 24, 24)], zs)
            for k in range(RWT // 24):
                pltpu.make_async_copy(
                    zbuf, acc.at[pl.ds(r0 + k * 24, 24)], zs).wait()
            plsc.subcore_barrier()

            for cx in (0, 1):
                nv, evc, rows, gs, isem = bufs[cx]
                idx_load(cx, nv, isem)
                idx_wait(nv, isem)
                decode_and_gather(nv, evc, rows, gs, g_hbm)
            idx_load(2, bufs[2][0], bufs[2][4])

            def body(q, _):
                for x in range(nbuf):
                    cx = nbuf * q + x
                    nv, evc, rows, gs, isem = bufs[x]
                    nv2, evc2, rows2, gs2, isem2 = bufs[(x + 2) % nbuf]
                    if x == 0:
                        @pl.when(q >= 1)
                        def _():
                            pltpu.make_async_copy(
                                rows2, acc.at[evc2], ss).wait()
                    else:
                        pltpu.make_async_copy(
                            bufs[x - 1][2], acc.at[bufs[x - 1][1]],
                            ss).wait()

                    @pl.when(cx + 2 < nbuf * qtrips)
                    def _():
                        idx_wait(nv2, isem2)
                        decode_and_gather(nv2, evc2, rows2, gs2, g_hbm)
                    pltpu.make_async_copy(g_hbm.at[nv], rows, gs).wait()

                    @pl.when(cx + nbuf < nbuf * qtrips)
                    def _():
                        idx_load(cx + nbuf, nv, isem)
                    pltpu.async_copy(rows, acc.at[evc], ss, add=True)
                return 0
            lax.fori_loop(0, qtrips, body, 0)
            pltpu.make_async_copy(rowsC, acc.at[evcC], ss).wait()
            plsc.subcore_barrier()

            pltpu.sync_copy(acc.at[pl.ds(r0, RWT)],
                            o_hbm.at[pl.ds(lo + r0, RWT)])
            if m == 0:
                plsc.subcore_barrier()

    return seg


_seg_s1 = _make_seg(False)
_seg_s2 = _make_seg(True)


def kernel(input, hg_idx, related_items, label, uid, params):
    p = params
    node = hg_idx[0]
    edge = hg_idx[1]

    pt, pi = _project_tables(p['text_table'], p['W1'], p['b1'],
                             p['img_table'], p['W2'], p['b2'])

    pos = jnp.asarray(_POS)
    ut_p = jnp.pad(p['user_table'], ((0, 0), (0, EMB - HALF)))
    (xt, xi, user, dinv, binv,
     pk_e, cnt_e, pk_n, cnt_n) = _sc_prep(
        pt, pi, related_items, node, edge, uid, ut_p)

    yt, yi = _layer_matmul(xt, p['theta_t0'], xi, p['theta_i0'], pos=pos)
    et_raw, ei_raw = _seg_s1(yt, yi, pk_e, cnt_e)
    et, ei = _escale(et_raw, ei_raw, binv)
    xt2, xi2 = _seg_s2(et, ei, pk_n, cnt_n)

    yt2, yi2 = _layer_matmul(xt2, p['theta_t1'], xi2, p['theta_i1'],
                             dinv=dinv, bt=p['bias_t0'], bi=p['bias_i0'])
    et2_raw, ei2_raw = _seg_s1(yt2, yi2, pk_e, cnt_e)
    et2, ei2 = _escale(et2_raw, ei2_raw, binv)
    xt3, xi3 = _seg_s2(et2, ei2, pk_n, cnt_n)
    tg0_raw = xt3[::LENS]
    ig0_raw = xi3[::LENS]

    d0 = dinv[::LENS].reshape(BSZ, 1)
    return _head(tg0_raw, ig0_raw, d0, p['bias_t1'], p['bias_i1'], user, p)
